# baseline (speedup 1.0000x reference)
import functools

import jax
import jax.numpy as jnp
import numpy as np
from jax import lax
from jax.experimental import pallas as pl
from jax.experimental.pallas import tpu as pltpu
from jax.experimental.pallas import tpu_sc as plsc

_INTERPRET = False

D = 128
N_ATOMS = 10000
N_BONDS = 319600
N_ANGLES = 319600
N_DIHEDRALS = 319600
NUM_GRAPHS = 800
N_LAYERS = 3
GAMMA = 10.0
_BL_CENTERS = np.arange(0.0, 2.0, 0.1).astype(np.float32)
_BA_CENTERS = np.arange(0.0, np.pi, 0.1).astype(np.float32)
_DA_CENTERS = np.arange(-np.pi, np.pi, 0.2).astype(np.float32)

_TILE = 512
_ATOM_PAD = 10240
_EDGE_PAD = 323584
_NC = 2
_NS = 16
_NW = _NC * _NS
_EPW = _EDGE_PAD // _NW
_ECHUNK = 128
_NCHUNKS = _EPW // _ECHUNK


def _pad_rows(a, n, value=0):
    return jnp.pad(a, ((0, n - a.shape[0]),) + ((0, 0),) * (a.ndim - 1),
                   constant_values=value)


def _embed_kernel(feats_ref, table_ref, centers_ref, out_ref, *, vocab, ncols):
    f = feats_ref[...]
    iota = jax.lax.broadcasted_iota(jnp.int32, (1, vocab), 1)
    blocks = [(f[:, j:j + 1] == iota).astype(jnp.float32) for j in range(ncols)]
    if centers_ref is not None:
        xs = jax.lax.bitcast_convert_type(f[:, ncols:ncols + 1], jnp.float32)
        blocks.append(jnp.exp(-GAMMA * (xs - centers_ref[...]) ** 2))
    oh = jnp.concatenate(blocks, axis=1)
    out_ref[...] = jnp.dot(oh, table_ref[...],
                           preferred_element_type=jnp.float32)


def _embed_call(feats_f32col, tables_stacked, vocab, ncols, centers, n_rows):
    grid = n_rows // _TILE
    has_c = centers is not None
    if has_c:
        kern = functools.partial(_embed_kernel, vocab=vocab, ncols=ncols)
    else:
        kern = functools.partial(
            lambda fr, tr, outr, **kw: _embed_kernel(fr, tr, None, outr, **kw),
            vocab=vocab, ncols=ncols)
    in_specs = [
        pl.BlockSpec((_TILE, feats_f32col.shape[1]), lambda i: (i, 0)),
        pl.BlockSpec(tables_stacked.shape, lambda i: (0, 0)),
    ]
    args = [feats_f32col, tables_stacked]
    if has_c:
        c = jnp.asarray(centers).reshape(1, -1)
        in_specs.append(pl.BlockSpec(c.shape, lambda i: (0, 0)))
        args.append(c)
    return pl.pallas_call(
        kern,
        grid=(grid,),
        in_specs=in_specs,
        out_specs=pl.BlockSpec((_TILE, D), lambda i: (i, 0)),
        out_shape=jax.ShapeDtypeStruct((n_rows, D), jnp.float32),
        interpret=_INTERPRET,
    )(*args)


def _rbf_kernel(x_ref, w_ref, b_ref, c_ref, out_ref):
    x = x_ref[...]
    feats = jnp.exp(-GAMMA * (x - c_ref[...]) ** 2)
    out_ref[...] = jnp.dot(feats, w_ref[...],
                           preferred_element_type=jnp.float32) + b_ref[...]


def _rbf_call(x, w, b, centers, n_rows):
    grid = n_rows // _TILE
    c = jnp.asarray(centers).reshape(1, -1)
    return pl.pallas_call(
        _rbf_kernel,
        grid=(grid,),
        in_specs=[
            pl.BlockSpec((_TILE, 1), lambda i: (i, 0)),
            pl.BlockSpec(w.shape, lambda i: (0, 0)),
            pl.BlockSpec((1, D), lambda i: (0, 0)),
            pl.BlockSpec(c.shape, lambda i: (0, 0)),
        ],
        out_specs=pl.BlockSpec((_TILE, D), lambda i: (i, 0)),
        out_shape=jax.ShapeDtypeStruct((n_rows, D), jnp.float32),
        interpret=_INTERPRET,
    )(x, w, b.reshape(1, D), c)


def _block_dense_kernel(agg_ref, resid_ref, batch_ref, invs_ref,
                        w1_ref, b1_ref, w2_ref, b2_ref, g_ref, bb_ref,
                        out_ref, *, act):
    agg = agg_ref[...]
    if agg.ndim == 3:
        agg = agg[0] + agg[1]
    h = jnp.dot(agg, w1_ref[...], preferred_element_type=jnp.float32) + b1_ref[...]
    h = jnp.maximum(h, 0.0)
    h = jnp.dot(h, w2_ref[...], preferred_element_type=jnp.float32) + b2_ref[...]
    mu = jnp.mean(h, axis=-1, keepdims=True)
    var = jnp.mean((h - mu) ** 2, axis=-1, keepdims=True)
    h = (h - mu) * jax.lax.rsqrt(var + 1e-5) * g_ref[...] + bb_ref[...]
    b = batch_ref[...]
    iota = jax.lax.broadcasted_iota(jnp.int32, (1, NUM_GRAPHS), 1)
    onehot = (b == iota).astype(jnp.float32)
    scale = jnp.dot(onehot, invs_ref[...],
                    preferred_element_type=jnp.float32)
    h = h * scale
    if act:
        h = jnp.maximum(h, 0.0)
    out_ref[...] = h + resid_ref[...]


def _block_dense_call(agg, resid, batch, invs_pg, bp, act, n_rows):
    grid = n_rows // _TILE
    kern = functools.partial(_block_dense_kernel, act=act)
    agg_spec = (pl.BlockSpec((_NC, _TILE, D), lambda i: (0, i, 0))
                if agg.ndim == 3 else pl.BlockSpec((_TILE, D), lambda i: (i, 0)))
    return pl.pallas_call(
        kern,
        grid=(grid,),
        in_specs=[
            agg_spec,
            pl.BlockSpec((_TILE, D), lambda i: (i, 0)),
            pl.BlockSpec((_TILE, 1), lambda i: (i, 0)),
            pl.BlockSpec((NUM_GRAPHS, 1), lambda i: (0, 0)),
            pl.BlockSpec((D, 2 * D), lambda i: (0, 0)),
            pl.BlockSpec((1, 2 * D), lambda i: (0, 0)),
            pl.BlockSpec((2 * D, D), lambda i: (0, 0)),
            pl.BlockSpec((1, D), lambda i: (0, 0)),
            pl.BlockSpec((1, D), lambda i: (0, 0)),
            pl.BlockSpec((1, D), lambda i: (0, 0)),
        ],
        out_specs=pl.BlockSpec((_TILE, D), lambda i: (i, 0)),
        out_shape=jax.ShapeDtypeStruct((n_rows, D), jnp.float32),
        interpret=_INTERPRET,
    )(agg, resid, batch, invs_pg,
      bp["W1"], bp["b1"].reshape(1, 2 * D), bp["W2"], bp["b2"].reshape(1, D),
      bp["ln_g"].reshape(1, D), bp["ln_b"].reshape(1, D))


def _graph_repr_kernel(node_ref, batch_ref, counts_ref, out_ref):
    i = pl.program_id(0)
    n = pl.num_programs(0)

    @pl.when(i == 0)
    def _init():
        out_ref[...] = jnp.zeros_like(out_ref)

    b = batch_ref[...].reshape(1, _TILE)
    iota = jax.lax.broadcasted_iota(jnp.int32, (NUM_GRAPHS, 1), 0)
    p = (iota == b).astype(jnp.float32)
    out_ref[...] += jnp.dot(p, node_ref[...],
                            preferred_element_type=jnp.float32)

    @pl.when(i == n - 1)
    def _fin():
        out_ref[...] = out_ref[...] / jnp.maximum(counts_ref[...], 1.0)


def _graph_repr_call(node_pad, batch_pad, counts_pg):
    grid = _ATOM_PAD // _TILE
    return pl.pallas_call(
        _graph_repr_kernel,
        grid=(grid,),
        in_specs=[
            pl.BlockSpec((_TILE, D), lambda i: (i, 0)),
            pl.BlockSpec((_TILE, 1), lambda i: (i, 0)),
            pl.BlockSpec((NUM_GRAPHS, 1), lambda i: (0, 0)),
        ],
        out_specs=pl.BlockSpec((NUM_GRAPHS, D), lambda i: (0, 0)),
        out_shape=jax.ShapeDtypeStruct((NUM_GRAPHS, D), jnp.float32),
        interpret=_INTERPRET,
    )(node_pad, batch_pad, counts_pg)


def _gather_segsum(node_hidden, edge_hidden, edges, num_nodes):
    src, dst = edges[0], edges[1]
    msg = jnp.take(node_hidden, src, axis=0) + edge_hidden
    return jax.ops.segment_sum(msg, dst, num_segments=num_nodes)


_LSZCAP = 329728
_FILL_CH = _LSZCAP // 16 // 128
_SCAT_CH = _EDGE_PAD // 16 // 128


def _count_kernel(dst_ref, out_ref, *, shift):
    i = pl.program_id(0)

    @pl.when(i == 0)
    def _():
        out_ref[...] = jnp.zeros_like(out_ref)

    bkt = lax.shift_right_logical(dst_ref[...], shift)
    iota = jax.lax.broadcasted_iota(jnp.int32, (1, 128), 1)
    oh = (bkt == iota).astype(jnp.float32)
    out_ref[...] += jnp.sum(oh, axis=0, keepdims=True)


def _count_call(dst_col, shift):
    kern = functools.partial(_count_kernel, shift=shift)
    return pl.pallas_call(
        kern,
        grid=(_EDGE_PAD // _TILE,),
        in_specs=[pl.BlockSpec((_TILE, 1), lambda i: (i, 0))],
        out_specs=pl.BlockSpec((8, 128), lambda i: (0, 0)),
        out_shape=jax.ShapeDtypeStruct((8, 128), jnp.float32),
        interpret=_INTERPRET,
    )(dst_col)


def _pos_kernel(dst_ref, bases_ref, pos_ref, run_ref, *, shift):
    i = pl.program_id(0)

    @pl.when(i == 0)
    def _():
        run_ref[...] = jnp.zeros_like(run_ref)

    bkt = lax.shift_right_logical(dst_ref[...], shift)
    iota = jax.lax.broadcasted_iota(jnp.int32, (1, 128), 1)
    oh = (bkt == iota).astype(jnp.float32)
    r = jax.lax.broadcasted_iota(jnp.int32, (_TILE, _TILE), 0)
    cc = jax.lax.broadcasted_iota(jnp.int32, (_TILE, _TILE), 1)
    ltri = (r > cc).astype(jnp.float32)
    prior = jnp.dot(ltri, oh, preferred_element_type=jnp.float32)
    br = bases_ref[0:1, :].astype(jnp.float32) + run_ref[0:1, :]
    pos = jnp.sum(oh * (prior + br), axis=1, keepdims=True)
    pos_ref[...] = pos.astype(jnp.int32)
    run_ref[...] += jnp.sum(oh, axis=0, keepdims=True)


def _pos_call(dst_col, bases, shift):
    kern = functools.partial(_pos_kernel, shift=shift)
    return pl.pallas_call(
        kern,
        grid=(_EDGE_PAD // _TILE,),
        in_specs=[pl.BlockSpec((_TILE, 1), lambda i: (i, 0)),
                  pl.BlockSpec((8, 128), lambda i: (0, 0))],
        out_specs=pl.BlockSpec((_TILE, 1), lambda i: (i, 0)),
        out_shape=jax.ShapeDtypeStruct((_EDGE_PAD, 1), jnp.int32),
        scratch_shapes=[pltpu.VMEM((8, 128), jnp.float32)],
        interpret=_INTERPRET,
    )(dst_col, bases)


def _sc_permute_body(pos_hbm, src_hbm, dst_hbm, perm_hbm, lsrc_hbm,
                     lldst_hbm, posv, sbuf, dbuf, lbuf, ebuf,
                     patv, pats, patl, s1, s2, s3, *, slab):
    c = lax.axis_index("c")
    s = lax.axis_index("s")

    @pl.when(c == 0)
    def _():
        iota = lax.iota(jnp.int32, 16)
        for k in range(8):
            patv[pl.ds(k * 16, 16)] = jnp.full((16,), -1, jnp.int32)
            pats[pl.ds(k * 16, 16)] = ((iota + k * 16) * 13) & 8191
            patl[pl.ds(k * 16, 16)] = slab + k * 16 + iota

        def fill(t, carry):
            off = pl.ds(s * (_FILL_CH * 128) + t * 128, 128)
            pltpu.sync_copy(patv, perm_hbm.at[off])
            pltpu.sync_copy(pats, lsrc_hbm.at[off])
            pltpu.sync_copy(patl, lldst_hbm.at[off])
            return carry

        lax.fori_loop(0, _FILL_CH, fill, 0)
        plsc.subcore_barrier()

        def scat(t, carry):
            st = s * (_SCAT_CH * 128) + t * 128
            pltpu.sync_copy(pos_hbm.at[pl.ds(st, 128)], posv)
            pltpu.sync_copy(src_hbm.at[pl.ds(st, 128)], sbuf)
            pltpu.sync_copy(dst_hbm.at[pl.ds(st, 128)], dbuf)
            iota = lax.iota(jnp.int32, 16)
            for k in range(8):
                ebuf[pl.ds(k * 16, 16)] = st + k * 16 + iota
                dv = dbuf[pl.ds(k * 16, 16)]
                lbuf[pl.ds(k * 16, 16)] = dv & (slab - 1)
            a1 = pltpu.async_copy(ebuf, perm_hbm.at[posv], s1)
            a2 = pltpu.async_copy(sbuf, lsrc_hbm.at[posv], s2)
            a3 = pltpu.async_copy(lbuf, lldst_hbm.at[posv], s3)
            a1.wait()
            a2.wait()
            a3.wait()
            return carry

        lax.fori_loop(0, _SCAT_CH, scat, 0)


def _sc_permute(pos_flat, src_flat, dst_flat, slab):
    body = functools.partial(_sc_permute_body, slab=slab)
    f = pl.kernel(
        body,
        out_type=[jax.ShapeDtypeStruct((_LSZCAP,), jnp.int32)] * 3,
        mesh=plsc.VectorSubcoreMesh(core_axis_name="c", subcore_axis_name="s"),
        scratch_types=[pltpu.VMEM((128,), jnp.int32)] * 8 + [
            pltpu.SemaphoreType.DMA,
            pltpu.SemaphoreType.DMA,
            pltpu.SemaphoreType.DMA,
        ],
    )
    return f(pos_flat, src_flat, dst_flat)


def _sc_consume_body(node_hbm, edge_hbm, lsrc_hbm, perm_hbm, lldst_hbm,
                     obs_hbm, obm_hbm, zeros_hbm, out_hbm,
                     osv, omv, idxv, permv, eidv, srcv, ldstv,
                     nbuf, ebuf, zbuf, accum,
                     s1, s2, s3, s4, s5, *, slab, npasses):
    c = lax.axis_index("c")
    s = lax.axis_index("s")
    pltpu.sync_copy(zeros_hbm, zbuf)
    rpt = slab // _NS
    iota = lax.iota(jnp.int32, 16)

    def pass_body(p, carry):
        b = p * _NC + c
        pltpu.sync_copy(obs_hbm.at[p, c], osv)
        pltpu.sync_copy(obm_hbm.at[p, c], omv)
        sv = osv[pl.ds(0, 16)]
        mv = omv[pl.ds(0, 16)]
        nch = mv[0]
        valid = mv[1]
        for k in range(rpt // 128):
            pltpu.sync_copy(zbuf, accum.at[pl.ds(s * rpt + k * 128, 128)])
        plsc.subcore_barrier()

        def chunk(jj, carry2):
            ch = s + jj * _NS
            for k in range(8):
                idxv[pl.ds(k * 16, 16)] = sv + ch * 128 + k * 16 + iota
            a1 = pltpu.async_copy(perm_hbm.at[idxv], permv, s1)
            a2 = pltpu.async_copy(lsrc_hbm.at[idxv], srcv, s2)
            a3 = pltpu.async_copy(lldst_hbm.at[idxv], ldstv, s3)
            a1.wait()
            for k in range(8):
                pv = permv[pl.ds(k * 16, 16)]
                eidv[pl.ds(k * 16, 16)] = jnp.where(
                    pv < 0, ((iota + k * 16) * 13) & 8191, pv)
            a4 = pltpu.async_copy(edge_hbm.at[eidv], ebuf, s4)
            a2.wait()
            a5 = pltpu.async_copy(node_hbm.at[srcv], nbuf, s5)
            a3.wait()
            a4.wait()
            a5.wait()
            w1 = pltpu.async_copy(nbuf, accum.at[ldstv], s4, add=True)
            w2 = pltpu.async_copy(ebuf, accum.at[ldstv], s5, add=True)
            w1.wait()
            w2.wait()
            return carry2

        ntr = jnp.maximum((nch - s + _NS - 1) // _NS, 0)
        lax.fori_loop(0, ntr, chunk, 0)
        plsc.subcore_barrier()

        @pl.when(valid > 0)
        def _():
            pltpu.sync_copy(
                accum.at[pl.ds(s * rpt, rpt)],
                out_hbm.at[pl.ds(b * slab + s * rpt, rpt)])

        return carry

    lax.fori_loop(0, npasses, pass_body, 0)


def _sc_consume(node_pad, edge_pad, lsrc, perm, lldst, obs, obm, nb, slab):
    npasses = (nb + _NC - 1) // _NC
    body = functools.partial(_sc_consume_body, slab=slab, npasses=npasses)
    f = pl.kernel(
        body,
        out_type=jax.ShapeDtypeStruct((nb * slab, D), jnp.float32),
        mesh=plsc.VectorSubcoreMesh(core_axis_name="c", subcore_axis_name="s"),
        scratch_types=[
            pltpu.VMEM((16,), jnp.int32),
            pltpu.VMEM((16,), jnp.int32),
            pltpu.VMEM((128,), jnp.int32),
            pltpu.VMEM((128,), jnp.int32),
            pltpu.VMEM((128,), jnp.int32),
            pltpu.VMEM((128,), jnp.int32),
            pltpu.VMEM((128,), jnp.int32),
            pltpu.VMEM((_ECHUNK, D), jnp.float32),
            pltpu.VMEM((_ECHUNK, D), jnp.float32),
            pltpu.VMEM((_ECHUNK, D), jnp.float32),
            pltpu.VMEM_SHARED((slab + _ECHUNK, D), jnp.float32),
            pltpu.SemaphoreType.DMA,
            pltpu.SemaphoreType.DMA,
            pltpu.SemaphoreType.DMA,
            pltpu.SemaphoreType.DMA,
            pltpu.SemaphoreType.DMA,
        ],
    )
    zeros128 = jnp.zeros((_ECHUNK, D), jnp.float32)
    return f(node_pad, edge_pad, lsrc, perm, lldst, obs, obm, zeros128)


def _graph_lists(src, dst, n_nodes, nb, shift):
    slab = 1 << shift
    npad = _EDGE_PAD - src.shape[0]
    ar = jnp.arange(npad, dtype=jnp.int32)
    src_f = jnp.concatenate([src.astype(jnp.int32), ar % n_nodes])
    dst_f = jnp.concatenate([dst.astype(jnp.int32),
                             n_nodes + ar % (nb * slab - n_nodes)])
    counts = _count_call(dst_f.reshape(-1, 1), shift)[0].astype(jnp.int32)
    seg = (counts + 127) & ~jnp.int32(127)
    off = jnp.concatenate([jnp.zeros((1,), jnp.int32), jnp.cumsum(seg)])
    bases = jnp.zeros((8, 128), jnp.int32).at[0, :65].set(off[:65])
    pos = _pos_call(dst_f.reshape(-1, 1), bases, shift)
    perm, lsrc, lldst = _sc_permute(pos.reshape(-1), src_f, dst_f, slab)
    npasses = (nb + _NC - 1) // _NC
    nch = (off[1:] - off[:-1]) // 128
    starts = off[:nb]
    nchb = nch[:nb]
    validb = jnp.ones((nb,), jnp.int32)
    if nb % _NC:
        starts = jnp.concatenate([starts, jnp.zeros((1,), jnp.int32)])
        nchb = jnp.concatenate([nchb, jnp.zeros((1,), jnp.int32)])
        validb = jnp.concatenate([validb, jnp.zeros((1,), jnp.int32)])
    obs = jnp.zeros((32, _NC, 16), jnp.int32)
    obs = obs.at[:npasses].set(
        jnp.broadcast_to(starts.reshape(npasses, _NC, 1), (npasses, _NC, 16)))
    obm = jnp.zeros((32, _NC, 16), jnp.int32)
    obm = obm.at[:npasses, :, 0].set(nchb.reshape(npasses, _NC))
    obm = obm.at[:npasses, :, 1].set(validb.reshape(npasses, _NC))
    return lsrc, perm, lldst, obs, obm


def kernel(AtomBondGraph_edges, BondAngleGraph_edges, AngleDihedralGraph_edges,
           x, bond_attr, bond_lengths, bond_angles, dihedral_angles,
           atom_batch, num_bonds, num_angles, num_graphs,
           masked_atom_indices, masked_bond_indices, masked_angle_indices,
           masked_dihedral_indices, params):
    _x = x.at[masked_atom_indices].set(15)
    _battr = bond_attr.at[masked_bond_indices].set(7)
    _bl = bond_lengths.at[masked_bond_indices].set(0.0)
    _ang = bond_angles.at[masked_angle_indices].set(0.0)
    _dih = dihedral_angles.at[masked_dihedral_indices].set(0.0)

    sb = jnp.searchsorted(atom_batch, jnp.arange(NUM_GRAPHS + 1, dtype=atom_batch.dtype))
    atom_counts = (sb[1:] - sb[:-1]).astype(jnp.float32)
    inv_atoms = jax.lax.rsqrt(jnp.maximum(atom_counts, 1.0)).reshape(NUM_GRAPHS, 1)
    bond_counts = num_bonds.astype(jnp.float32)
    inv_bonds = jax.lax.rsqrt(jnp.maximum(bond_counts, 1.0)).reshape(NUM_GRAPHS, 1)
    angle_counts = num_angles.astype(jnp.float32)
    inv_angles = jax.lax.rsqrt(jnp.maximum(angle_counts, 1.0)).reshape(NUM_GRAPHS, 1)

    gid = jnp.arange(NUM_GRAPHS, dtype=jnp.int32)
    bond_batch = jnp.repeat(gid, num_bonds, total_repeat_length=N_BONDS)
    angle_batch = jnp.repeat(gid, num_angles, total_repeat_length=N_ANGLES)
    ab_pad = _pad_rows(atom_batch.astype(jnp.int32).reshape(-1, 1), _ATOM_PAD, -1)
    bb_pad = _pad_rows(bond_batch.reshape(-1, 1), _EDGE_PAD, -1)
    anb_pad = _pad_rows(angle_batch.reshape(-1, 1), _EDGE_PAD, -1)

    atom_tab = jnp.concatenate(params["atom_emb"], axis=0)
    x_pad = _pad_rows(_x.astype(jnp.int32), _ATOM_PAD)
    node_hidden = _embed_call(x_pad, atom_tab, 16, 7, None, _ATOM_PAD)[:N_ATOMS]

    def bond_feat_input():
        battr_pad = _pad_rows(_battr.astype(jnp.int32), _EDGE_PAD)
        bl_bits = jax.lax.bitcast_convert_type(
            _bl.astype(jnp.float32), jnp.int32).reshape(-1, 1)
        return jnp.concatenate([battr_pad, _pad_rows(bl_bits, _EDGE_PAD)], axis=1)

    bond_feats = bond_feat_input()

    def bond_embed(tables, rbf_p):
        tab = jnp.concatenate(list(tables) + [rbf_p["W"]], axis=0)
        out = _embed_call(bond_feats, tab, 8, 3, _BL_CENTERS, _EDGE_PAD)
        return out + rbf_p["b"][None, :]

    bond_hidden = bond_embed(params["init_bond_emb"], params["init_bond_rbf"])[:N_BONDS]

    ang_pad = _pad_rows(_ang.astype(jnp.float32), _EDGE_PAD)
    dih_pad = _pad_rows(_dih.astype(jnp.float32), _EDGE_PAD)
    angle_hidden = _rbf_call(ang_pad, params["init_angle_rbf"]["W"],
                             params["init_angle_rbf"]["b"], _BA_CENTERS,
                             _EDGE_PAD)[:N_ANGLES]

    ab_g = _graph_lists(AtomBondGraph_edges[0], AtomBondGraph_edges[1],
                        N_ATOMS, 3, 12)
    ba_g = _graph_lists(BondAngleGraph_edges[0], BondAngleGraph_edges[1],
                        N_BONDS, 40, 13)
    ad_g = _graph_lists(AngleDihedralGraph_edges[0], AngleDihedralGraph_edges[1],
                        N_ANGLES, 40, 13)

    nh_pad = _pad_rows(node_hidden, _ATOM_PAD)
    eh_pad = _pad_rows(bond_hidden, _EDGE_PAD)
    ah_pad = _pad_rows(angle_hidden, _EDGE_PAD)
    dih_hidden = None
    for lid in range(N_LAYERS):
        lp = params["layers"][lid]
        act = lid != N_LAYERS - 1

        agg_a = _sc_consume(nh_pad, eh_pad, ab_g[0], ab_g[1], ab_g[2],
                            ab_g[3], ab_g[4], 3, 4096)
        nh_pad = _block_dense_call(
            agg_a, nh_pad, ab_pad, inv_atoms, lp["ab_block"], act, _ATOM_PAD)

        cur_edge_pad = bond_embed(lp["bond_emb"], lp["bond_rbf"])
        agg_b = _sc_consume(cur_edge_pad, ah_pad, ba_g[0], ba_g[1], ba_g[2],
                            ba_g[3], ba_g[4], 40, 8192)
        eh_pad = _block_dense_call(
            agg_b, cur_edge_pad, bb_pad,
            inv_bonds, lp["ba_block"], act, _EDGE_PAD)

        cur_angle_pad = _rbf_call(ang_pad, lp["angle_rbf"]["W"],
                                  lp["angle_rbf"]["b"], _BA_CENTERS, _EDGE_PAD)
        dih_hidden = _rbf_call(dih_pad, lp["dihedral_rbf"]["W"],
                               lp["dihedral_rbf"]["b"], _DA_CENTERS, _EDGE_PAD)
        agg_an = _sc_consume(cur_angle_pad, dih_hidden, ad_g[0], ad_g[1],
                             ad_g[2], ad_g[3], ad_g[4], 40, 8192)
        ah_pad = _block_dense_call(
            agg_an, cur_angle_pad,
            anb_pad, inv_angles, lp["ad_block"], act, _EDGE_PAD)

    graph_repr = _graph_repr_call(nh_pad, ab_pad,
                                  atom_counts.reshape(NUM_GRAPHS, 1))
    return (nh_pad[:N_ATOMS], eh_pad[:N_BONDS], ah_pad[:N_ANGLES],
            dih_hidden[:N_DIHEDRALS], graph_repr)


# speedup vs baseline: 1.0042x; 1.0042x over previous
import functools

import jax
import jax.numpy as jnp
import numpy as np
from jax import lax
from jax.experimental import pallas as pl
from jax.experimental.pallas import tpu as pltpu
from jax.experimental.pallas import tpu_sc as plsc

_INTERPRET = False

D = 128
N_ATOMS = 10000
N_BONDS = 319600
N_ANGLES = 319600
N_DIHEDRALS = 319600
NUM_GRAPHS = 800
N_LAYERS = 3
GAMMA = 10.0
_BL_CENTERS = np.arange(0.0, 2.0, 0.1).astype(np.float32)
_BA_CENTERS = np.arange(0.0, np.pi, 0.1).astype(np.float32)
_DA_CENTERS = np.arange(-np.pi, np.pi, 0.2).astype(np.float32)

_TILE = 512
_ATOM_PAD = 10240
_EDGE_PAD = 323584
_NC = 2
_NS = 16
_NW = _NC * _NS
_EPW = _EDGE_PAD // _NW
_ECHUNK = 128
_NCHUNKS = _EPW // _ECHUNK


def _pad_rows(a, n, value=0):
    return jnp.pad(a, ((0, n - a.shape[0]),) + ((0, 0),) * (a.ndim - 1),
                   constant_values=value)


def _embed_kernel(feats_ref, table_ref, centers_ref, out_ref, *, vocab, ncols):
    f = feats_ref[...]
    iota = jax.lax.broadcasted_iota(jnp.int32, (1, vocab), 1)
    blocks = [(f[:, j:j + 1] == iota).astype(jnp.float32) for j in range(ncols)]
    if centers_ref is not None:
        xs = jax.lax.bitcast_convert_type(f[:, ncols:ncols + 1], jnp.float32)
        blocks.append(jnp.exp(-GAMMA * (xs - centers_ref[...]) ** 2))
    oh = jnp.concatenate(blocks, axis=1)
    out_ref[...] = jnp.dot(oh, table_ref[...],
                           preferred_element_type=jnp.float32)


def _embed_call(feats_f32col, tables_stacked, vocab, ncols, centers, n_rows):
    grid = n_rows // _TILE
    has_c = centers is not None
    if has_c:
        kern = functools.partial(_embed_kernel, vocab=vocab, ncols=ncols)
    else:
        kern = functools.partial(
            lambda fr, tr, outr, **kw: _embed_kernel(fr, tr, None, outr, **kw),
            vocab=vocab, ncols=ncols)
    in_specs = [
        pl.BlockSpec((_TILE, feats_f32col.shape[1]), lambda i: (i, 0)),
        pl.BlockSpec(tables_stacked.shape, lambda i: (0, 0)),
    ]
    args = [feats_f32col, tables_stacked]
    if has_c:
        c = jnp.asarray(centers).reshape(1, -1)
        in_specs.append(pl.BlockSpec(c.shape, lambda i: (0, 0)))
        args.append(c)
    return pl.pallas_call(
        kern,
        grid=(grid,),
        in_specs=in_specs,
        out_specs=pl.BlockSpec((_TILE, D), lambda i: (i, 0)),
        out_shape=jax.ShapeDtypeStruct((n_rows, D), jnp.float32),
        interpret=_INTERPRET,
    )(*args)


def _rbf_kernel(x_ref, w_ref, b_ref, c_ref, out_ref):
    x = x_ref[...]
    feats = jnp.exp(-GAMMA * (x - c_ref[...]) ** 2)
    out_ref[...] = jnp.dot(feats, w_ref[...],
                           preferred_element_type=jnp.float32) + b_ref[...]


def _rbf_call(x, w, b, centers, n_rows):
    grid = n_rows // _TILE
    c = jnp.asarray(centers).reshape(1, -1)
    return pl.pallas_call(
        _rbf_kernel,
        grid=(grid,),
        in_specs=[
            pl.BlockSpec((_TILE, 1), lambda i: (i, 0)),
            pl.BlockSpec(w.shape, lambda i: (0, 0)),
            pl.BlockSpec((1, D), lambda i: (0, 0)),
            pl.BlockSpec(c.shape, lambda i: (0, 0)),
        ],
        out_specs=pl.BlockSpec((_TILE, D), lambda i: (i, 0)),
        out_shape=jax.ShapeDtypeStruct((n_rows, D), jnp.float32),
        interpret=_INTERPRET,
    )(x, w, b.reshape(1, D), c)


def _block_dense_kernel(agg_ref, resid_ref, batch_ref, invs_ref,
                        w1_ref, b1_ref, w2_ref, b2_ref, g_ref, bb_ref,
                        out_ref, *, act):
    agg = agg_ref[...]
    if agg.ndim == 3:
        agg = agg[0] + agg[1]
    h = jnp.dot(agg, w1_ref[...], preferred_element_type=jnp.float32) + b1_ref[...]
    h = jnp.maximum(h, 0.0)
    h = jnp.dot(h, w2_ref[...], preferred_element_type=jnp.float32) + b2_ref[...]
    mu = jnp.mean(h, axis=-1, keepdims=True)
    var = jnp.mean((h - mu) ** 2, axis=-1, keepdims=True)
    h = (h - mu) * jax.lax.rsqrt(var + 1e-5) * g_ref[...] + bb_ref[...]
    b = batch_ref[...]
    iota = jax.lax.broadcasted_iota(jnp.int32, (1, NUM_GRAPHS), 1)
    onehot = (b == iota).astype(jnp.float32)
    scale = jnp.dot(onehot, invs_ref[...],
                    preferred_element_type=jnp.float32)
    h = h * scale
    if act:
        h = jnp.maximum(h, 0.0)
    out_ref[...] = h + resid_ref[...]


def _block_dense_call(agg, resid, batch, invs_pg, bp, act, n_rows):
    grid = n_rows // _TILE
    kern = functools.partial(_block_dense_kernel, act=act)
    agg_spec = (pl.BlockSpec((_NC, _TILE, D), lambda i: (0, i, 0))
                if agg.ndim == 3 else pl.BlockSpec((_TILE, D), lambda i: (i, 0)))
    return pl.pallas_call(
        kern,
        grid=(grid,),
        in_specs=[
            agg_spec,
            pl.BlockSpec((_TILE, D), lambda i: (i, 0)),
            pl.BlockSpec((_TILE, 1), lambda i: (i, 0)),
            pl.BlockSpec((NUM_GRAPHS, 1), lambda i: (0, 0)),
            pl.BlockSpec((D, 2 * D), lambda i: (0, 0)),
            pl.BlockSpec((1, 2 * D), lambda i: (0, 0)),
            pl.BlockSpec((2 * D, D), lambda i: (0, 0)),
            pl.BlockSpec((1, D), lambda i: (0, 0)),
            pl.BlockSpec((1, D), lambda i: (0, 0)),
            pl.BlockSpec((1, D), lambda i: (0, 0)),
        ],
        out_specs=pl.BlockSpec((_TILE, D), lambda i: (i, 0)),
        out_shape=jax.ShapeDtypeStruct((n_rows, D), jnp.float32),
        interpret=_INTERPRET,
    )(agg, resid, batch, invs_pg,
      bp["W1"], bp["b1"].reshape(1, 2 * D), bp["W2"], bp["b2"].reshape(1, D),
      bp["ln_g"].reshape(1, D), bp["ln_b"].reshape(1, D))


def _graph_repr_kernel(node_ref, batch_ref, counts_ref, out_ref):
    i = pl.program_id(0)
    n = pl.num_programs(0)

    @pl.when(i == 0)
    def _init():
        out_ref[...] = jnp.zeros_like(out_ref)

    b = batch_ref[...].reshape(1, _TILE)
    iota = jax.lax.broadcasted_iota(jnp.int32, (NUM_GRAPHS, 1), 0)
    p = (iota == b).astype(jnp.float32)
    out_ref[...] += jnp.dot(p, node_ref[...],
                            preferred_element_type=jnp.float32)

    @pl.when(i == n - 1)
    def _fin():
        out_ref[...] = out_ref[...] / jnp.maximum(counts_ref[...], 1.0)


def _graph_repr_call(node_pad, batch_pad, counts_pg):
    grid = _ATOM_PAD // _TILE
    return pl.pallas_call(
        _graph_repr_kernel,
        grid=(grid,),
        in_specs=[
            pl.BlockSpec((_TILE, D), lambda i: (i, 0)),
            pl.BlockSpec((_TILE, 1), lambda i: (i, 0)),
            pl.BlockSpec((NUM_GRAPHS, 1), lambda i: (0, 0)),
        ],
        out_specs=pl.BlockSpec((NUM_GRAPHS, D), lambda i: (0, 0)),
        out_shape=jax.ShapeDtypeStruct((NUM_GRAPHS, D), jnp.float32),
        interpret=_INTERPRET,
    )(node_pad, batch_pad, counts_pg)


def _gather_segsum(node_hidden, edge_hidden, edges, num_nodes):
    src, dst = edges[0], edges[1]
    msg = jnp.take(node_hidden, src, axis=0) + edge_hidden
    return jax.ops.segment_sum(msg, dst, num_segments=num_nodes)


_LSZCAP = 335872
_FILL_CH = _LSZCAP // 16 // 128
_SCAT_CH = _EDGE_PAD // 16 // 128


def _count_kernel(dst_ref, out_ref, *, shift):
    i = pl.program_id(0)

    @pl.when(i == 0)
    def _():
        out_ref[...] = jnp.zeros_like(out_ref)

    bkt = lax.shift_right_logical(dst_ref[...], shift)
    iota = jax.lax.broadcasted_iota(jnp.int32, (1, 128), 1)
    oh = (bkt == iota).astype(jnp.float32)
    out_ref[...] += jnp.sum(oh, axis=0, keepdims=True)


def _count_call(dst_col, shift):
    kern = functools.partial(_count_kernel, shift=shift)
    return pl.pallas_call(
        kern,
        grid=(_EDGE_PAD // _TILE,),
        in_specs=[pl.BlockSpec((_TILE, 1), lambda i: (i, 0))],
        out_specs=pl.BlockSpec((8, 128), lambda i: (0, 0)),
        out_shape=jax.ShapeDtypeStruct((8, 128), jnp.float32),
        interpret=_INTERPRET,
    )(dst_col)


def _pos_kernel(dst_ref, bases_ref, pos_ref, run_ref, *, shift):
    i = pl.program_id(0)

    @pl.when(i == 0)
    def _():
        run_ref[...] = jnp.zeros_like(run_ref)

    bkt = lax.shift_right_logical(dst_ref[...], shift)
    iota = jax.lax.broadcasted_iota(jnp.int32, (1, 128), 1)
    oh = (bkt == iota).astype(jnp.float32)
    r = jax.lax.broadcasted_iota(jnp.int32, (_TILE, _TILE), 0)
    cc = jax.lax.broadcasted_iota(jnp.int32, (_TILE, _TILE), 1)
    ltri = (r > cc).astype(jnp.float32)
    prior = jnp.dot(ltri, oh, preferred_element_type=jnp.float32)
    br = bases_ref[0:1, :].astype(jnp.float32) + run_ref[0:1, :]
    pos = jnp.sum(oh * (prior + br), axis=1, keepdims=True)
    pos_ref[...] = pos.astype(jnp.int32)
    run_ref[...] += jnp.sum(oh, axis=0, keepdims=True)


def _pos_call(dst_col, bases, shift):
    kern = functools.partial(_pos_kernel, shift=shift)
    return pl.pallas_call(
        kern,
        grid=(_EDGE_PAD // _TILE,),
        in_specs=[pl.BlockSpec((_TILE, 1), lambda i: (i, 0)),
                  pl.BlockSpec((8, 128), lambda i: (0, 0))],
        out_specs=pl.BlockSpec((_TILE, 1), lambda i: (i, 0)),
        out_shape=jax.ShapeDtypeStruct((_EDGE_PAD, 1), jnp.int32),
        scratch_shapes=[pltpu.VMEM((8, 128), jnp.float32)],
        interpret=_INTERPRET,
    )(dst_col, bases)


def _sc_permute_body(pos_hbm, src_hbm, dst_hbm, perm_hbm, lsrc_hbm,
                     lldst_hbm, posv, sbuf, dbuf, lbuf, ebuf,
                     patv, pats, patl, s1, s2, s3, *, slab):
    c = lax.axis_index("c")
    s = lax.axis_index("s")

    @pl.when(c == 0)
    def _():
        iota = lax.iota(jnp.int32, 16)
        for k in range(8):
            patv[pl.ds(k * 16, 16)] = jnp.full((16,), -1, jnp.int32)
            pats[pl.ds(k * 16, 16)] = ((iota + k * 16) * 13) & 8191
            patl[pl.ds(k * 16, 16)] = slab + k * 16 + iota

        def fill(t, carry):
            off = pl.ds(s * (_FILL_CH * 128) + t * 128, 128)
            pltpu.sync_copy(patv, perm_hbm.at[off])
            pltpu.sync_copy(pats, lsrc_hbm.at[off])
            pltpu.sync_copy(patl, lldst_hbm.at[off])
            return carry

        lax.fori_loop(0, _FILL_CH, fill, 0)
        plsc.subcore_barrier()

        def scat(t, carry):
            st = s * (_SCAT_CH * 128) + t * 128
            pltpu.sync_copy(pos_hbm.at[pl.ds(st, 128)], posv)
            pltpu.sync_copy(src_hbm.at[pl.ds(st, 128)], sbuf)
            pltpu.sync_copy(dst_hbm.at[pl.ds(st, 128)], dbuf)
            iota = lax.iota(jnp.int32, 16)
            for k in range(8):
                ebuf[pl.ds(k * 16, 16)] = st + k * 16 + iota
                dv = dbuf[pl.ds(k * 16, 16)]
                lbuf[pl.ds(k * 16, 16)] = dv & (slab - 1)
            a1 = pltpu.async_copy(ebuf, perm_hbm.at[posv], s1)
            a2 = pltpu.async_copy(sbuf, lsrc_hbm.at[posv], s2)
            a3 = pltpu.async_copy(lbuf, lldst_hbm.at[posv], s3)
            a1.wait()
            a2.wait()
            a3.wait()
            return carry

        lax.fori_loop(0, _SCAT_CH, scat, 0)


def _sc_permute(pos_flat, src_flat, dst_flat, slab):
    body = functools.partial(_sc_permute_body, slab=slab)
    f = pl.kernel(
        body,
        out_type=[jax.ShapeDtypeStruct((_LSZCAP,), jnp.int32)] * 3,
        mesh=plsc.VectorSubcoreMesh(core_axis_name="c", subcore_axis_name="s"),
        scratch_types=[pltpu.VMEM((128,), jnp.int32)] * 8 + [
            pltpu.SemaphoreType.DMA,
            pltpu.SemaphoreType.DMA,
            pltpu.SemaphoreType.DMA,
        ],
    )
    return f(pos_flat, src_flat, dst_flat)


def _sc_consume_body(node_hbm, edge_hbm, lsrc_hbm, perm_hbm, lldst_hbm,
                     obs_hbm, obm_hbm, zeros_hbm, out_hbm,
                     osv, omv, idxv, permv, eidv, srcv, ldstv_a, ldstv_b,
                     nbuf, ebuf, zbuf, accum,
                     s1, s2, s3, s4, s5, *, slab, npasses):
    c = lax.axis_index("c")
    s = lax.axis_index("s")
    pltpu.sync_copy(zeros_hbm, zbuf)
    rpt = slab // _NS
    iota = lax.iota(jnp.int32, 16)

    def pass_body(p, carry):
        b = p * _NC + c
        pltpu.sync_copy(obs_hbm.at[p, c], osv)
        pltpu.sync_copy(obm_hbm.at[p, c], omv)
        sv = osv[pl.ds(0, 16)]
        mv = omv[pl.ds(0, 16)]
        nch = mv[0]
        valid = mv[1]
        for k in range(rpt // 128):
            pltpu.sync_copy(zbuf, accum.at[pl.ds(s * rpt + k * 128, 128)])
        plsc.subcore_barrier()

        def chunk(jj, carry2):
            ch = s + jj * _NS
            for k in range(16):
                idxv[pl.ds(k * 16, 16)] = sv + ch * 256 + k * 16 + iota
            a1 = pltpu.async_copy(perm_hbm.at[idxv], permv, s1)
            a2 = pltpu.async_copy(lsrc_hbm.at[idxv], srcv, s2)
            a3 = pltpu.async_copy(
                lldst_hbm.at[idxv.at[pl.ds(0, 128)]], ldstv_a, s3)
            a4 = pltpu.async_copy(
                lldst_hbm.at[idxv.at[pl.ds(128, 128)]], ldstv_b, s3)
            a1.wait()
            for k in range(16):
                pv = permv[pl.ds(k * 16, 16)]
                eidv[pl.ds(k * 16, 16)] = jnp.where(
                    pv < 0, ((iota + k * 16) * 13) & 8191, pv)
            a2.wait()
            a3.wait()
            a4.wait()
            for h in range(2):
                lv = ldstv_a if h == 0 else ldstv_b
                g1 = pltpu.async_copy(
                    node_hbm.at[srcv.at[pl.ds(h * 128, 128)]], nbuf, s4)
                g2 = pltpu.async_copy(
                    edge_hbm.at[eidv.at[pl.ds(h * 128, 128)]], ebuf, s5)
                g1.wait()
                g2.wait()
                w1 = pltpu.async_copy(nbuf, accum.at[lv], s4, add=True)
                w2 = pltpu.async_copy(ebuf, accum.at[lv], s5, add=True)
                w1.wait()
                w2.wait()
            return carry2

        ntr = jnp.maximum((nch - s + _NS - 1) // _NS, 0)
        lax.fori_loop(0, ntr, chunk, 0)
        plsc.subcore_barrier()

        @pl.when(valid > 0)
        def _():
            pltpu.sync_copy(
                accum.at[pl.ds(s * rpt, rpt)],
                out_hbm.at[pl.ds(b * slab + s * rpt, rpt)])

        return carry

    lax.fori_loop(0, npasses, pass_body, 0)


def _sc_consume(node_pad, edge_pad, lsrc, perm, lldst, obs, obm, nb, slab):
    npasses = (nb + _NC - 1) // _NC
    body = functools.partial(_sc_consume_body, slab=slab, npasses=npasses)
    f = pl.kernel(
        body,
        out_type=jax.ShapeDtypeStruct((nb * slab, D), jnp.float32),
        mesh=plsc.VectorSubcoreMesh(core_axis_name="c", subcore_axis_name="s"),
        scratch_types=[
            pltpu.VMEM((16,), jnp.int32),
            pltpu.VMEM((16,), jnp.int32),
            pltpu.VMEM((256,), jnp.int32),
            pltpu.VMEM((256,), jnp.int32),
            pltpu.VMEM((256,), jnp.int32),
            pltpu.VMEM((256,), jnp.int32),
            pltpu.VMEM((128,), jnp.int32),
            pltpu.VMEM((128,), jnp.int32),
            pltpu.VMEM((_ECHUNK, D), jnp.float32),
            pltpu.VMEM((_ECHUNK, D), jnp.float32),
            pltpu.VMEM((_ECHUNK, D), jnp.float32),
            pltpu.VMEM_SHARED((slab + _ECHUNK, D), jnp.float32),
            pltpu.SemaphoreType.DMA,
            pltpu.SemaphoreType.DMA,
            pltpu.SemaphoreType.DMA,
            pltpu.SemaphoreType.DMA,
            pltpu.SemaphoreType.DMA,
        ],
    )
    zeros128 = jnp.zeros((_ECHUNK, D), jnp.float32)
    return f(node_pad, edge_pad, lsrc, perm, lldst, obs, obm, zeros128)


def _graph_lists(src, dst, n_nodes, nb, shift):
    slab = 1 << shift
    npad = _EDGE_PAD - src.shape[0]
    ar = jnp.arange(npad, dtype=jnp.int32)
    src_f = jnp.concatenate([src.astype(jnp.int32), ar % n_nodes])
    dst_f = jnp.concatenate([dst.astype(jnp.int32),
                             n_nodes + ar % (nb * slab - n_nodes)])
    counts = _count_call(dst_f.reshape(-1, 1), shift)[0].astype(jnp.int32)
    seg = (counts + 255) & ~jnp.int32(255)
    off = jnp.concatenate([jnp.zeros((1,), jnp.int32), jnp.cumsum(seg)])
    bases = jnp.zeros((8, 128), jnp.int32).at[0, :65].set(off[:65])
    pos = _pos_call(dst_f.reshape(-1, 1), bases, shift)
    perm, lsrc, lldst = _sc_permute(pos.reshape(-1), src_f, dst_f, slab)
    npasses = (nb + _NC - 1) // _NC
    nch = (off[1:] - off[:-1]) // 256
    starts = off[:nb]
    nchb = nch[:nb]
    validb = jnp.ones((nb,), jnp.int32)
    if nb % _NC:
        starts = jnp.concatenate([starts, jnp.zeros((1,), jnp.int32)])
        nchb = jnp.concatenate([nchb, jnp.zeros((1,), jnp.int32)])
        validb = jnp.concatenate([validb, jnp.zeros((1,), jnp.int32)])
    obs = jnp.zeros((32, _NC, 16), jnp.int32)
    obs = obs.at[:npasses].set(
        jnp.broadcast_to(starts.reshape(npasses, _NC, 1), (npasses, _NC, 16)))
    obm = jnp.zeros((32, _NC, 16), jnp.int32)
    obm = obm.at[:npasses, :, 0].set(nchb.reshape(npasses, _NC))
    obm = obm.at[:npasses, :, 1].set(validb.reshape(npasses, _NC))
    return lsrc, perm, lldst, obs, obm


def kernel(AtomBondGraph_edges, BondAngleGraph_edges, AngleDihedralGraph_edges,
           x, bond_attr, bond_lengths, bond_angles, dihedral_angles,
           atom_batch, num_bonds, num_angles, num_graphs,
           masked_atom_indices, masked_bond_indices, masked_angle_indices,
           masked_dihedral_indices, params):
    _x = x.at[masked_atom_indices].set(15)
    _battr = bond_attr.at[masked_bond_indices].set(7)
    _bl = bond_lengths.at[masked_bond_indices].set(0.0)
    _ang = bond_angles.at[masked_angle_indices].set(0.0)
    _dih = dihedral_angles.at[masked_dihedral_indices].set(0.0)

    sb = jnp.searchsorted(atom_batch, jnp.arange(NUM_GRAPHS + 1, dtype=atom_batch.dtype))
    atom_counts = (sb[1:] - sb[:-1]).astype(jnp.float32)
    inv_atoms = jax.lax.rsqrt(jnp.maximum(atom_counts, 1.0)).reshape(NUM_GRAPHS, 1)
    bond_counts = num_bonds.astype(jnp.float32)
    inv_bonds = jax.lax.rsqrt(jnp.maximum(bond_counts, 1.0)).reshape(NUM_GRAPHS, 1)
    angle_counts = num_angles.astype(jnp.float32)
    inv_angles = jax.lax.rsqrt(jnp.maximum(angle_counts, 1.0)).reshape(NUM_GRAPHS, 1)

    gid = jnp.arange(NUM_GRAPHS, dtype=jnp.int32)
    bond_batch = jnp.repeat(gid, num_bonds, total_repeat_length=N_BONDS)
    angle_batch = jnp.repeat(gid, num_angles, total_repeat_length=N_ANGLES)
    ab_pad = _pad_rows(atom_batch.astype(jnp.int32).reshape(-1, 1), _ATOM_PAD, -1)
    bb_pad = _pad_rows(bond_batch.reshape(-1, 1), _EDGE_PAD, -1)
    anb_pad = _pad_rows(angle_batch.reshape(-1, 1), _EDGE_PAD, -1)

    atom_tab = jnp.concatenate(params["atom_emb"], axis=0)
    x_pad = _pad_rows(_x.astype(jnp.int32), _ATOM_PAD)
    node_hidden = _embed_call(x_pad, atom_tab, 16, 7, None, _ATOM_PAD)[:N_ATOMS]

    def bond_feat_input():
        battr_pad = _pad_rows(_battr.astype(jnp.int32), _EDGE_PAD)
        bl_bits = jax.lax.bitcast_convert_type(
            _bl.astype(jnp.float32), jnp.int32).reshape(-1, 1)
        return jnp.concatenate([battr_pad, _pad_rows(bl_bits, _EDGE_PAD)], axis=1)

    bond_feats = bond_feat_input()

    def bond_embed(tables, rbf_p):
        tab = jnp.concatenate(list(tables) + [rbf_p["W"]], axis=0)
        out = _embed_call(bond_feats, tab, 8, 3, _BL_CENTERS, _EDGE_PAD)
        return out + rbf_p["b"][None, :]

    bond_hidden = bond_embed(params["init_bond_emb"], params["init_bond_rbf"])[:N_BONDS]

    ang_pad = _pad_rows(_ang.astype(jnp.float32), _EDGE_PAD)
    dih_pad = _pad_rows(_dih.astype(jnp.float32), _EDGE_PAD)
    angle_hidden = _rbf_call(ang_pad, params["init_angle_rbf"]["W"],
                             params["init_angle_rbf"]["b"], _BA_CENTERS,
                             _EDGE_PAD)[:N_ANGLES]

    ab_g = _graph_lists(AtomBondGraph_edges[0], AtomBondGraph_edges[1],
                        N_ATOMS, 3, 12)
    ba_g = _graph_lists(BondAngleGraph_edges[0], BondAngleGraph_edges[1],
                        N_BONDS, 40, 13)
    ad_g = _graph_lists(AngleDihedralGraph_edges[0], AngleDihedralGraph_edges[1],
                        N_ANGLES, 40, 13)

    nh_pad = _pad_rows(node_hidden, _ATOM_PAD)
    eh_pad = _pad_rows(bond_hidden, _EDGE_PAD)
    ah_pad = _pad_rows(angle_hidden, _EDGE_PAD)
    dih_hidden = None
    for lid in range(N_LAYERS):
        lp = params["layers"][lid]
        act = lid != N_LAYERS - 1

        agg_a = _sc_consume(nh_pad, eh_pad, ab_g[0], ab_g[1], ab_g[2],
                            ab_g[3], ab_g[4], 3, 4096)
        nh_pad = _block_dense_call(
            agg_a, nh_pad, ab_pad, inv_atoms, lp["ab_block"], act, _ATOM_PAD)

        cur_edge_pad = bond_embed(lp["bond_emb"], lp["bond_rbf"])
        agg_b = _sc_consume(cur_edge_pad, ah_pad, ba_g[0], ba_g[1], ba_g[2],
                            ba_g[3], ba_g[4], 40, 8192)
        eh_pad = _block_dense_call(
            agg_b, cur_edge_pad, bb_pad,
            inv_bonds, lp["ba_block"], act, _EDGE_PAD)

        cur_angle_pad = _rbf_call(ang_pad, lp["angle_rbf"]["W"],
                                  lp["angle_rbf"]["b"], _BA_CENTERS, _EDGE_PAD)
        dih_hidden = _rbf_call(dih_pad, lp["dihedral_rbf"]["W"],
                               lp["dihedral_rbf"]["b"], _DA_CENTERS, _EDGE_PAD)
        agg_an = _sc_consume(cur_angle_pad, dih_hidden, ad_g[0], ad_g[1],
                             ad_g[2], ad_g[3], ad_g[4], 40, 8192)
        ah_pad = _block_dense_call(
            agg_an, cur_angle_pad,
            anb_pad, inv_angles, lp["ad_block"], act, _EDGE_PAD)

    graph_repr = _graph_repr_call(nh_pad, ab_pad,
                                  atom_counts.reshape(NUM_GRAPHS, 1))
    return (nh_pad[:N_ATOMS], eh_pad[:N_BONDS], ah_pad[:N_ANGLES],
            dih_hidden[:N_DIHEDRALS], graph_repr)


# speedup vs baseline: 1.0042x; 1.0000x over previous
import functools

import jax
import jax.numpy as jnp
import numpy as np
from jax import lax
from jax.experimental import pallas as pl
from jax.experimental.pallas import tpu as pltpu
from jax.experimental.pallas import tpu_sc as plsc

_INTERPRET = False

D = 128
N_ATOMS = 10000
N_BONDS = 319600
N_ANGLES = 319600
N_DIHEDRALS = 319600
NUM_GRAPHS = 800
N_LAYERS = 3
GAMMA = 10.0
_BL_CENTERS = np.arange(0.0, 2.0, 0.1).astype(np.float32)
_BA_CENTERS = np.arange(0.0, np.pi, 0.1).astype(np.float32)
_DA_CENTERS = np.arange(-np.pi, np.pi, 0.2).astype(np.float32)

_TILE = 512
_ATOM_PAD = 10240
_EDGE_PAD = 323584
_NC = 2
_NS = 16
_NW = _NC * _NS
_EPW = _EDGE_PAD // _NW
_ECHUNK = 128
_NCHUNKS = _EPW // _ECHUNK


def _pad_rows(a, n, value=0):
    return jnp.pad(a, ((0, n - a.shape[0]),) + ((0, 0),) * (a.ndim - 1),
                   constant_values=value)


def _embed_kernel(feats_ref, table_ref, centers_ref, out_ref, *, vocab, ncols):
    f = feats_ref[...]
    iota = jax.lax.broadcasted_iota(jnp.int32, (1, vocab), 1)
    blocks = [(f[:, j:j + 1] == iota).astype(jnp.float32) for j in range(ncols)]
    if centers_ref is not None:
        xs = jax.lax.bitcast_convert_type(f[:, ncols:ncols + 1], jnp.float32)
        blocks.append(jnp.exp(-GAMMA * (xs - centers_ref[...]) ** 2))
    oh = jnp.concatenate(blocks, axis=1)
    out_ref[...] = jnp.dot(oh, table_ref[...],
                           preferred_element_type=jnp.float32)


def _embed_call(feats_f32col, tables_stacked, vocab, ncols, centers, n_rows):
    grid = n_rows // _TILE
    has_c = centers is not None
    if has_c:
        kern = functools.partial(_embed_kernel, vocab=vocab, ncols=ncols)
    else:
        kern = functools.partial(
            lambda fr, tr, outr, **kw: _embed_kernel(fr, tr, None, outr, **kw),
            vocab=vocab, ncols=ncols)
    in_specs = [
        pl.BlockSpec((_TILE, feats_f32col.shape[1]), lambda i: (i, 0)),
        pl.BlockSpec(tables_stacked.shape, lambda i: (0, 0)),
    ]
    args = [feats_f32col, tables_stacked]
    if has_c:
        c = jnp.asarray(centers).reshape(1, -1)
        in_specs.append(pl.BlockSpec(c.shape, lambda i: (0, 0)))
        args.append(c)
    return pl.pallas_call(
        kern,
        grid=(grid,),
        in_specs=in_specs,
        out_specs=pl.BlockSpec((_TILE, D), lambda i: (i, 0)),
        out_shape=jax.ShapeDtypeStruct((n_rows, D), jnp.float32),
        interpret=_INTERPRET,
    )(*args)


def _rbf_kernel(x_ref, w_ref, b_ref, c_ref, out_ref):
    x = x_ref[...]
    feats = jnp.exp(-GAMMA * (x - c_ref[...]) ** 2)
    out_ref[...] = jnp.dot(feats, w_ref[...],
                           preferred_element_type=jnp.float32) + b_ref[...]


def _rbf_call(x, w, b, centers, n_rows):
    grid = n_rows // _TILE
    c = jnp.asarray(centers).reshape(1, -1)
    return pl.pallas_call(
        _rbf_kernel,
        grid=(grid,),
        in_specs=[
            pl.BlockSpec((_TILE, 1), lambda i: (i, 0)),
            pl.BlockSpec(w.shape, lambda i: (0, 0)),
            pl.BlockSpec((1, D), lambda i: (0, 0)),
            pl.BlockSpec(c.shape, lambda i: (0, 0)),
        ],
        out_specs=pl.BlockSpec((_TILE, D), lambda i: (i, 0)),
        out_shape=jax.ShapeDtypeStruct((n_rows, D), jnp.float32),
        interpret=_INTERPRET,
    )(x, w, b.reshape(1, D), c)


def _block_dense_kernel(agg_ref, resid_ref, batch_ref, invs_ref,
                        w1_ref, b1_ref, w2_ref, b2_ref, g_ref, bb_ref,
                        out_ref, *, act):
    agg = agg_ref[...]
    if agg.ndim == 3:
        agg = agg[0] + agg[1]
    h = jnp.dot(agg, w1_ref[...], preferred_element_type=jnp.float32) + b1_ref[...]
    h = jnp.maximum(h, 0.0)
    h = jnp.dot(h, w2_ref[...], preferred_element_type=jnp.float32) + b2_ref[...]
    mu = jnp.mean(h, axis=-1, keepdims=True)
    var = jnp.mean((h - mu) ** 2, axis=-1, keepdims=True)
    h = (h - mu) * jax.lax.rsqrt(var + 1e-5) * g_ref[...] + bb_ref[...]
    b = batch_ref[...]
    iota = jax.lax.broadcasted_iota(jnp.int32, (1, NUM_GRAPHS), 1)
    onehot = (b == iota).astype(jnp.float32)
    scale = jnp.dot(onehot, invs_ref[...],
                    preferred_element_type=jnp.float32)
    h = h * scale
    if act:
        h = jnp.maximum(h, 0.0)
    out_ref[...] = h + resid_ref[...]


def _block_dense_call(agg, resid, batch, invs_pg, bp, act, n_rows):
    grid = n_rows // _TILE
    kern = functools.partial(_block_dense_kernel, act=act)
    agg_spec = (pl.BlockSpec((_NC, _TILE, D), lambda i: (0, i, 0))
                if agg.ndim == 3 else pl.BlockSpec((_TILE, D), lambda i: (i, 0)))
    return pl.pallas_call(
        kern,
        grid=(grid,),
        in_specs=[
            agg_spec,
            pl.BlockSpec((_TILE, D), lambda i: (i, 0)),
            pl.BlockSpec((_TILE, 1), lambda i: (i, 0)),
            pl.BlockSpec((NUM_GRAPHS, 1), lambda i: (0, 0)),
            pl.BlockSpec((D, 2 * D), lambda i: (0, 0)),
            pl.BlockSpec((1, 2 * D), lambda i: (0, 0)),
            pl.BlockSpec((2 * D, D), lambda i: (0, 0)),
            pl.BlockSpec((1, D), lambda i: (0, 0)),
            pl.BlockSpec((1, D), lambda i: (0, 0)),
            pl.BlockSpec((1, D), lambda i: (0, 0)),
        ],
        out_specs=pl.BlockSpec((_TILE, D), lambda i: (i, 0)),
        out_shape=jax.ShapeDtypeStruct((n_rows, D), jnp.float32),
        interpret=_INTERPRET,
    )(agg, resid, batch, invs_pg,
      bp["W1"], bp["b1"].reshape(1, 2 * D), bp["W2"], bp["b2"].reshape(1, D),
      bp["ln_g"].reshape(1, D), bp["ln_b"].reshape(1, D))


def _graph_repr_kernel(node_ref, batch_ref, counts_ref, out_ref):
    i = pl.program_id(0)
    n = pl.num_programs(0)

    @pl.when(i == 0)
    def _init():
        out_ref[...] = jnp.zeros_like(out_ref)

    b = batch_ref[...].reshape(1, _TILE)
    iota = jax.lax.broadcasted_iota(jnp.int32, (NUM_GRAPHS, 1), 0)
    p = (iota == b).astype(jnp.float32)
    out_ref[...] += jnp.dot(p, node_ref[...],
                            preferred_element_type=jnp.float32)

    @pl.when(i == n - 1)
    def _fin():
        out_ref[...] = out_ref[...] / jnp.maximum(counts_ref[...], 1.0)


def _graph_repr_call(node_pad, batch_pad, counts_pg):
    grid = _ATOM_PAD // _TILE
    return pl.pallas_call(
        _graph_repr_kernel,
        grid=(grid,),
        in_specs=[
            pl.BlockSpec((_TILE, D), lambda i: (i, 0)),
            pl.BlockSpec((_TILE, 1), lambda i: (i, 0)),
            pl.BlockSpec((NUM_GRAPHS, 1), lambda i: (0, 0)),
        ],
        out_specs=pl.BlockSpec((NUM_GRAPHS, D), lambda i: (0, 0)),
        out_shape=jax.ShapeDtypeStruct((NUM_GRAPHS, D), jnp.float32),
        interpret=_INTERPRET,
    )(node_pad, batch_pad, counts_pg)


def _gather_segsum(node_hidden, edge_hidden, edges, num_nodes):
    src, dst = edges[0], edges[1]
    msg = jnp.take(node_hidden, src, axis=0) + edge_hidden
    return jax.ops.segment_sum(msg, dst, num_segments=num_nodes)


_LSZCAP = 335872
_FILL_CH = _LSZCAP // 16 // 128
_SCAT_CH = _EDGE_PAD // 16 // 128


def _count_kernel(dst_ref, out_ref, *, shift):
    i = pl.program_id(0)

    @pl.when(i == 0)
    def _():
        out_ref[...] = jnp.zeros_like(out_ref)

    bkt = lax.shift_right_logical(dst_ref[...], shift)
    iota = jax.lax.broadcasted_iota(jnp.int32, (1, 128), 1)
    oh = (bkt == iota).astype(jnp.float32)
    out_ref[...] += jnp.sum(oh, axis=0, keepdims=True)


def _count_call(dst_col, shift):
    kern = functools.partial(_count_kernel, shift=shift)
    return pl.pallas_call(
        kern,
        grid=(_EDGE_PAD // _TILE,),
        in_specs=[pl.BlockSpec((_TILE, 1), lambda i: (i, 0))],
        out_specs=pl.BlockSpec((8, 128), lambda i: (0, 0)),
        out_shape=jax.ShapeDtypeStruct((8, 128), jnp.float32),
        interpret=_INTERPRET,
    )(dst_col)


def _pos_kernel(dst_ref, bases_ref, pos_ref, run_ref, *, shift):
    i = pl.program_id(0)

    @pl.when(i == 0)
    def _():
        run_ref[...] = jnp.zeros_like(run_ref)

    bkt = lax.shift_right_logical(dst_ref[...], shift)
    iota = jax.lax.broadcasted_iota(jnp.int32, (1, 128), 1)
    oh = (bkt == iota).astype(jnp.float32)
    r = jax.lax.broadcasted_iota(jnp.int32, (_TILE, _TILE), 0)
    cc = jax.lax.broadcasted_iota(jnp.int32, (_TILE, _TILE), 1)
    ltri = (r > cc).astype(jnp.float32)
    prior = jnp.dot(ltri, oh, preferred_element_type=jnp.float32)
    br = bases_ref[0:1, :].astype(jnp.float32) + run_ref[0:1, :]
    pos = jnp.sum(oh * (prior + br), axis=1, keepdims=True)
    pos_ref[...] = pos.astype(jnp.int32)
    run_ref[...] += jnp.sum(oh, axis=0, keepdims=True)


def _pos_call(dst_col, bases, shift):
    kern = functools.partial(_pos_kernel, shift=shift)
    return pl.pallas_call(
        kern,
        grid=(_EDGE_PAD // _TILE,),
        in_specs=[pl.BlockSpec((_TILE, 1), lambda i: (i, 0)),
                  pl.BlockSpec((8, 128), lambda i: (0, 0))],
        out_specs=pl.BlockSpec((_TILE, 1), lambda i: (i, 0)),
        out_shape=jax.ShapeDtypeStruct((_EDGE_PAD, 1), jnp.int32),
        scratch_shapes=[pltpu.VMEM((8, 128), jnp.float32)],
        interpret=_INTERPRET,
    )(dst_col, bases)


def _sc_permute_body(pos_hbm, src_hbm, dst_hbm, perm_hbm, lsrc_hbm,
                     lldst_hbm, posv, sbuf, dbuf, lbuf, ebuf,
                     patv, pats, patl, s1, s2, s3, *, slab):
    c = lax.axis_index("c")
    s = lax.axis_index("s")

    @pl.when(c == 0)
    def _():
        iota = lax.iota(jnp.int32, 16)
        for k in range(8):
            patv[pl.ds(k * 16, 16)] = jnp.full((16,), -1, jnp.int32)
            pats[pl.ds(k * 16, 16)] = ((iota + k * 16) * 13) & 8191
            patl[pl.ds(k * 16, 16)] = slab + k * 16 + iota

        def fill(t, carry):
            off = pl.ds(s * (_FILL_CH * 128) + t * 128, 128)
            pltpu.sync_copy(patv, perm_hbm.at[off])
            pltpu.sync_copy(pats, lsrc_hbm.at[off])
            pltpu.sync_copy(patl, lldst_hbm.at[off])
            return carry

        lax.fori_loop(0, _FILL_CH, fill, 0)
        plsc.subcore_barrier()

        def scat(t, carry):
            st = s * (_SCAT_CH * 128) + t * 128
            pltpu.sync_copy(pos_hbm.at[pl.ds(st, 128)], posv)
            pltpu.sync_copy(src_hbm.at[pl.ds(st, 128)], sbuf)
            pltpu.sync_copy(dst_hbm.at[pl.ds(st, 128)], dbuf)
            iota = lax.iota(jnp.int32, 16)
            for k in range(8):
                ebuf[pl.ds(k * 16, 16)] = st + k * 16 + iota
                dv = dbuf[pl.ds(k * 16, 16)]
                lbuf[pl.ds(k * 16, 16)] = dv & (slab - 1)
            a1 = pltpu.async_copy(ebuf, perm_hbm.at[posv], s1)
            a2 = pltpu.async_copy(sbuf, lsrc_hbm.at[posv], s2)
            a3 = pltpu.async_copy(lbuf, lldst_hbm.at[posv], s3)
            a1.wait()
            a2.wait()
            a3.wait()
            return carry

        lax.fori_loop(0, _SCAT_CH, scat, 0)


def _sc_permute(pos_flat, src_flat, dst_flat, slab):
    body = functools.partial(_sc_permute_body, slab=slab)
    f = pl.kernel(
        body,
        out_type=[jax.ShapeDtypeStruct((_LSZCAP,), jnp.int32)] * 3,
        mesh=plsc.VectorSubcoreMesh(core_axis_name="c", subcore_axis_name="s"),
        scratch_types=[pltpu.VMEM((128,), jnp.int32)] * 8 + [
            pltpu.SemaphoreType.DMA,
            pltpu.SemaphoreType.DMA,
            pltpu.SemaphoreType.DMA,
        ],
    )
    return f(pos_flat, src_flat, dst_flat)


def _sc_consume_body(node_hbm, edge_hbm, lsrc_hbm, perm_hbm, lldst_hbm,
                     obs_hbm, obm_hbm, zeros_hbm, out_hbm,
                     osv, omv, idxv, permv, eidv, srcv, ldstv_a, ldstv_b,
                     nbuf, ebuf, zbuf, accum,
                     s1, s2, s3, s4, s5, *, slab, npasses):
    c = lax.axis_index("c")
    s = lax.axis_index("s")
    pltpu.sync_copy(zeros_hbm, zbuf)
    rpt = slab // _NS
    iota = lax.iota(jnp.int32, 16)

    def pass_body(p, carry):
        b = p * _NC + c
        pltpu.sync_copy(obs_hbm.at[p, c], osv)
        pltpu.sync_copy(obm_hbm.at[p, c], omv)
        sv = osv[pl.ds(0, 16)]
        mv = omv[pl.ds(0, 16)]
        nch = mv[0]
        valid = mv[1]
        for k in range(rpt // 128):
            pltpu.sync_copy(zbuf, accum.at[pl.ds(s * rpt + k * 128, 128)])
        plsc.subcore_barrier()

        def chunk(jj, carry2):
            ch = s + jj * _NS
            for k in range(16):
                idxv[pl.ds(k * 16, 16)] = sv + ch * 256 + k * 16 + iota
            a1 = pltpu.async_copy(perm_hbm.at[idxv], permv, s1)
            a2 = pltpu.async_copy(lsrc_hbm.at[idxv], srcv, s2)
            a3 = pltpu.async_copy(
                lldst_hbm.at[idxv.at[pl.ds(0, 128)]], ldstv_a, s3)
            a4 = pltpu.async_copy(
                lldst_hbm.at[idxv.at[pl.ds(128, 128)]], ldstv_b, s3)
            a1.wait()
            for k in range(16):
                pv = permv[pl.ds(k * 16, 16)]
                eidv[pl.ds(k * 16, 16)] = jnp.where(
                    pv < 0, ((iota + k * 16) * 13) & 8191, pv)
            a2.wait()
            a3.wait()
            a4.wait()
            g1 = pltpu.async_copy(
                node_hbm.at[srcv.at[pl.ds(0, 128)]], nbuf, s4)
            g2 = pltpu.async_copy(
                node_hbm.at[srcv.at[pl.ds(128, 128)]], ebuf, s5)
            g1.wait()
            g3 = pltpu.async_copy(
                edge_hbm.at[eidv.at[pl.ds(0, 128)]], nbuf, s4, add=True)
            g2.wait()
            g4 = pltpu.async_copy(
                edge_hbm.at[eidv.at[pl.ds(128, 128)]], ebuf, s5, add=True)
            g3.wait()
            w1 = pltpu.async_copy(nbuf, accum.at[ldstv_a], s4, add=True)
            g4.wait()
            w2 = pltpu.async_copy(ebuf, accum.at[ldstv_b], s5, add=True)
            w1.wait()
            w2.wait()
            return carry2

        ntr = jnp.maximum((nch - s + _NS - 1) // _NS, 0)
        lax.fori_loop(0, ntr, chunk, 0)
        plsc.subcore_barrier()

        @pl.when(valid > 0)
        def _():
            pltpu.sync_copy(
                accum.at[pl.ds(s * rpt, rpt)],
                out_hbm.at[pl.ds(b * slab + s * rpt, rpt)])

        return carry

    lax.fori_loop(0, npasses, pass_body, 0)


def _sc_consume(node_pad, edge_pad, lsrc, perm, lldst, obs, obm, nb, slab):
    npasses = (nb + _NC - 1) // _NC
    body = functools.partial(_sc_consume_body, slab=slab, npasses=npasses)
    f = pl.kernel(
        body,
        out_type=jax.ShapeDtypeStruct((nb * slab, D), jnp.float32),
        mesh=plsc.VectorSubcoreMesh(core_axis_name="c", subcore_axis_name="s"),
        scratch_types=[
            pltpu.VMEM((16,), jnp.int32),
            pltpu.VMEM((16,), jnp.int32),
            pltpu.VMEM((256,), jnp.int32),
            pltpu.VMEM((256,), jnp.int32),
            pltpu.VMEM((256,), jnp.int32),
            pltpu.VMEM((256,), jnp.int32),
            pltpu.VMEM((128,), jnp.int32),
            pltpu.VMEM((128,), jnp.int32),
            pltpu.VMEM((_ECHUNK, D), jnp.float32),
            pltpu.VMEM((_ECHUNK, D), jnp.float32),
            pltpu.VMEM((_ECHUNK, D), jnp.float32),
            pltpu.VMEM_SHARED((slab + _ECHUNK, D), jnp.float32),
            pltpu.SemaphoreType.DMA,
            pltpu.SemaphoreType.DMA,
            pltpu.SemaphoreType.DMA,
            pltpu.SemaphoreType.DMA,
            pltpu.SemaphoreType.DMA,
        ],
    )
    zeros128 = jnp.zeros((_ECHUNK, D), jnp.float32)
    return f(node_pad, edge_pad, lsrc, perm, lldst, obs, obm, zeros128)


def _graph_lists(src, dst, n_nodes, nb, shift):
    slab = 1 << shift
    npad = _EDGE_PAD - src.shape[0]
    ar = jnp.arange(npad, dtype=jnp.int32)
    src_f = jnp.concatenate([src.astype(jnp.int32), ar % n_nodes])
    dst_f = jnp.concatenate([dst.astype(jnp.int32),
                             n_nodes + ar % (nb * slab - n_nodes)])
    counts = _count_call(dst_f.reshape(-1, 1), shift)[0].astype(jnp.int32)
    seg = (counts + 255) & ~jnp.int32(255)
    off = jnp.concatenate([jnp.zeros((1,), jnp.int32), jnp.cumsum(seg)])
    bases = jnp.zeros((8, 128), jnp.int32).at[0, :65].set(off[:65])
    pos = _pos_call(dst_f.reshape(-1, 1), bases, shift)
    perm, lsrc, lldst = _sc_permute(pos.reshape(-1), src_f, dst_f, slab)
    npasses = (nb + _NC - 1) // _NC
    nch = (off[1:] - off[:-1]) // 256
    starts = off[:nb]
    nchb = nch[:nb]
    validb = jnp.ones((nb,), jnp.int32)
    if nb % _NC:
        starts = jnp.concatenate([starts, jnp.zeros((1,), jnp.int32)])
        nchb = jnp.concatenate([nchb, jnp.zeros((1,), jnp.int32)])
        validb = jnp.concatenate([validb, jnp.zeros((1,), jnp.int32)])
    obs = jnp.zeros((32, _NC, 16), jnp.int32)
    obs = obs.at[:npasses].set(
        jnp.broadcast_to(starts.reshape(npasses, _NC, 1), (npasses, _NC, 16)))
    obm = jnp.zeros((32, _NC, 16), jnp.int32)
    obm = obm.at[:npasses, :, 0].set(nchb.reshape(npasses, _NC))
    obm = obm.at[:npasses, :, 1].set(validb.reshape(npasses, _NC))
    return lsrc, perm, lldst, obs, obm


def kernel(AtomBondGraph_edges, BondAngleGraph_edges, AngleDihedralGraph_edges,
           x, bond_attr, bond_lengths, bond_angles, dihedral_angles,
           atom_batch, num_bonds, num_angles, num_graphs,
           masked_atom_indices, masked_bond_indices, masked_angle_indices,
           masked_dihedral_indices, params):
    _x = x.at[masked_atom_indices].set(15)
    _battr = bond_attr.at[masked_bond_indices].set(7)
    _bl = bond_lengths.at[masked_bond_indices].set(0.0)
    _ang = bond_angles.at[masked_angle_indices].set(0.0)
    _dih = dihedral_angles.at[masked_dihedral_indices].set(0.0)

    sb = jnp.searchsorted(atom_batch, jnp.arange(NUM_GRAPHS + 1, dtype=atom_batch.dtype))
    atom_counts = (sb[1:] - sb[:-1]).astype(jnp.float32)
    inv_atoms = jax.lax.rsqrt(jnp.maximum(atom_counts, 1.0)).reshape(NUM_GRAPHS, 1)
    bond_counts = num_bonds.astype(jnp.float32)
    inv_bonds = jax.lax.rsqrt(jnp.maximum(bond_counts, 1.0)).reshape(NUM_GRAPHS, 1)
    angle_counts = num_angles.astype(jnp.float32)
    inv_angles = jax.lax.rsqrt(jnp.maximum(angle_counts, 1.0)).reshape(NUM_GRAPHS, 1)

    gid = jnp.arange(NUM_GRAPHS, dtype=jnp.int32)
    bond_batch = jnp.repeat(gid, num_bonds, total_repeat_length=N_BONDS)
    angle_batch = jnp.repeat(gid, num_angles, total_repeat_length=N_ANGLES)
    ab_pad = _pad_rows(atom_batch.astype(jnp.int32).reshape(-1, 1), _ATOM_PAD, -1)
    bb_pad = _pad_rows(bond_batch.reshape(-1, 1), _EDGE_PAD, -1)
    anb_pad = _pad_rows(angle_batch.reshape(-1, 1), _EDGE_PAD, -1)

    atom_tab = jnp.concatenate(params["atom_emb"], axis=0)
    x_pad = _pad_rows(_x.astype(jnp.int32), _ATOM_PAD)
    node_hidden = _embed_call(x_pad, atom_tab, 16, 7, None, _ATOM_PAD)[:N_ATOMS]

    def bond_feat_input():
        battr_pad = _pad_rows(_battr.astype(jnp.int32), _EDGE_PAD)
        bl_bits = jax.lax.bitcast_convert_type(
            _bl.astype(jnp.float32), jnp.int32).reshape(-1, 1)
        return jnp.concatenate([battr_pad, _pad_rows(bl_bits, _EDGE_PAD)], axis=1)

    bond_feats = bond_feat_input()

    def bond_embed(tables, rbf_p):
        tab = jnp.concatenate(list(tables) + [rbf_p["W"]], axis=0)
        out = _embed_call(bond_feats, tab, 8, 3, _BL_CENTERS, _EDGE_PAD)
        return out + rbf_p["b"][None, :]

    bond_hidden = bond_embed(params["init_bond_emb"], params["init_bond_rbf"])[:N_BONDS]

    ang_pad = _pad_rows(_ang.astype(jnp.float32), _EDGE_PAD)
    dih_pad = _pad_rows(_dih.astype(jnp.float32), _EDGE_PAD)
    angle_hidden = _rbf_call(ang_pad, params["init_angle_rbf"]["W"],
                             params["init_angle_rbf"]["b"], _BA_CENTERS,
                             _EDGE_PAD)[:N_ANGLES]

    ab_g = _graph_lists(AtomBondGraph_edges[0], AtomBondGraph_edges[1],
                        N_ATOMS, 3, 12)
    ba_g = _graph_lists(BondAngleGraph_edges[0], BondAngleGraph_edges[1],
                        N_BONDS, 40, 13)
    ad_g = _graph_lists(AngleDihedralGraph_edges[0], AngleDihedralGraph_edges[1],
                        N_ANGLES, 40, 13)

    nh_pad = _pad_rows(node_hidden, _ATOM_PAD)
    eh_pad = _pad_rows(bond_hidden, _EDGE_PAD)
    ah_pad = _pad_rows(angle_hidden, _EDGE_PAD)
    dih_hidden = None
    for lid in range(N_LAYERS):
        lp = params["layers"][lid]
        act = lid != N_LAYERS - 1

        agg_a = _sc_consume(nh_pad, eh_pad, ab_g[0], ab_g[1], ab_g[2],
                            ab_g[3], ab_g[4], 3, 4096)
        nh_pad = _block_dense_call(
            agg_a, nh_pad, ab_pad, inv_atoms, lp["ab_block"], act, _ATOM_PAD)

        cur_edge_pad = bond_embed(lp["bond_emb"], lp["bond_rbf"])
        agg_b = _sc_consume(cur_edge_pad, ah_pad, ba_g[0], ba_g[1], ba_g[2],
                            ba_g[3], ba_g[4], 40, 8192)
        eh_pad = _block_dense_call(
            agg_b, cur_edge_pad, bb_pad,
            inv_bonds, lp["ba_block"], act, _EDGE_PAD)

        cur_angle_pad = _rbf_call(ang_pad, lp["angle_rbf"]["W"],
                                  lp["angle_rbf"]["b"], _BA_CENTERS, _EDGE_PAD)
        dih_hidden = _rbf_call(dih_pad, lp["dihedral_rbf"]["W"],
                               lp["dihedral_rbf"]["b"], _DA_CENTERS, _EDGE_PAD)
        agg_an = _sc_consume(cur_angle_pad, dih_hidden, ad_g[0], ad_g[1],
                             ad_g[2], ad_g[3], ad_g[4], 40, 8192)
        ah_pad = _block_dense_call(
            agg_an, cur_angle_pad,
            anb_pad, inv_angles, lp["ad_block"], act, _EDGE_PAD)

    graph_repr = _graph_repr_call(nh_pad, ab_pad,
                                  atom_counts.reshape(NUM_GRAPHS, 1))
    return (nh_pad[:N_ATOMS], eh_pad[:N_BONDS], ah_pad[:N_ANGLES],
            dih_hidden[:N_DIHEDRALS], graph_repr)


# speedup vs baseline: 1.0049x; 1.0007x over previous
import functools

import jax
import jax.numpy as jnp
import numpy as np
from jax import lax
from jax.experimental import pallas as pl
from jax.experimental.pallas import tpu as pltpu
from jax.experimental.pallas import tpu_sc as plsc

_INTERPRET = False

D = 128
N_ATOMS = 10000
N_BONDS = 319600
N_ANGLES = 319600
N_DIHEDRALS = 319600
NUM_GRAPHS = 800
N_LAYERS = 3
GAMMA = 10.0
_BL_CENTERS = np.arange(0.0, 2.0, 0.1).astype(np.float32)
_BA_CENTERS = np.arange(0.0, np.pi, 0.1).astype(np.float32)
_DA_CENTERS = np.arange(-np.pi, np.pi, 0.2).astype(np.float32)

_TILE = 512
_ATOM_PAD = 10240
_EDGE_PAD = 323584
_NC = 2
_NS = 16
_NW = _NC * _NS
_EPW = _EDGE_PAD // _NW
_ECHUNK = 128
_NCHUNKS = _EPW // _ECHUNK


def _pad_rows(a, n, value=0):
    return jnp.pad(a, ((0, n - a.shape[0]),) + ((0, 0),) * (a.ndim - 1),
                   constant_values=value)


def _embed_kernel(feats_ref, table_ref, centers_ref, out_ref, *, vocab, ncols):
    f = feats_ref[...]
    iota = jax.lax.broadcasted_iota(jnp.int32, (1, vocab), 1)
    blocks = [(f[:, j:j + 1] == iota).astype(jnp.float32) for j in range(ncols)]
    if centers_ref is not None:
        xs = jax.lax.bitcast_convert_type(f[:, ncols:ncols + 1], jnp.float32)
        blocks.append(jnp.exp(-GAMMA * (xs - centers_ref[...]) ** 2))
    oh = jnp.concatenate(blocks, axis=1)
    out_ref[...] = jnp.dot(oh, table_ref[...],
                           preferred_element_type=jnp.float32)


def _embed_call(feats_f32col, tables_stacked, vocab, ncols, centers, n_rows):
    grid = n_rows // _TILE
    has_c = centers is not None
    if has_c:
        kern = functools.partial(_embed_kernel, vocab=vocab, ncols=ncols)
    else:
        kern = functools.partial(
            lambda fr, tr, outr, **kw: _embed_kernel(fr, tr, None, outr, **kw),
            vocab=vocab, ncols=ncols)
    in_specs = [
        pl.BlockSpec((_TILE, feats_f32col.shape[1]), lambda i: (i, 0)),
        pl.BlockSpec(tables_stacked.shape, lambda i: (0, 0)),
    ]
    args = [feats_f32col, tables_stacked]
    if has_c:
        c = jnp.asarray(centers).reshape(1, -1)
        in_specs.append(pl.BlockSpec(c.shape, lambda i: (0, 0)))
        args.append(c)
    return pl.pallas_call(
        kern,
        grid=(grid,),
        in_specs=in_specs,
        out_specs=pl.BlockSpec((_TILE, D), lambda i: (i, 0)),
        out_shape=jax.ShapeDtypeStruct((n_rows, D), jnp.float32),
        interpret=_INTERPRET,
    )(*args)


def _rbf_kernel(x_ref, w_ref, b_ref, c_ref, out_ref):
    x = x_ref[...]
    feats = jnp.exp(-GAMMA * (x - c_ref[...]) ** 2)
    out_ref[...] = jnp.dot(feats, w_ref[...],
                           preferred_element_type=jnp.float32) + b_ref[...]


def _rbf_call(x, w, b, centers, n_rows):
    grid = n_rows // _TILE
    c = jnp.asarray(centers).reshape(1, -1)
    return pl.pallas_call(
        _rbf_kernel,
        grid=(grid,),
        in_specs=[
            pl.BlockSpec((_TILE, 1), lambda i: (i, 0)),
            pl.BlockSpec(w.shape, lambda i: (0, 0)),
            pl.BlockSpec((1, D), lambda i: (0, 0)),
            pl.BlockSpec(c.shape, lambda i: (0, 0)),
        ],
        out_specs=pl.BlockSpec((_TILE, D), lambda i: (i, 0)),
        out_shape=jax.ShapeDtypeStruct((n_rows, D), jnp.float32),
        interpret=_INTERPRET,
    )(x, w, b.reshape(1, D), c)


def _block_dense_kernel(agg_ref, resid_ref, batch_ref, invs_ref,
                        w1_ref, b1_ref, w2_ref, b2_ref, g_ref, bb_ref,
                        out_ref, *, act):
    agg = agg_ref[...]
    if agg.ndim == 3:
        agg = agg[0] + agg[1]
    h = jnp.dot(agg, w1_ref[...], preferred_element_type=jnp.float32) + b1_ref[...]
    h = jnp.maximum(h, 0.0)
    h = jnp.dot(h, w2_ref[...], preferred_element_type=jnp.float32) + b2_ref[...]
    mu = jnp.mean(h, axis=-1, keepdims=True)
    var = jnp.mean((h - mu) ** 2, axis=-1, keepdims=True)
    h = (h - mu) * jax.lax.rsqrt(var + 1e-5) * g_ref[...] + bb_ref[...]
    b = batch_ref[...]
    iota = jax.lax.broadcasted_iota(jnp.int32, (1, NUM_GRAPHS), 1)
    onehot = (b == iota).astype(jnp.float32)
    scale = jnp.dot(onehot, invs_ref[...],
                    preferred_element_type=jnp.float32)
    h = h * scale
    if act:
        h = jnp.maximum(h, 0.0)
    out_ref[...] = h + resid_ref[...]


def _block_dense_call(agg, resid, batch, invs_pg, bp, act, n_rows):
    grid = n_rows // _TILE
    kern = functools.partial(_block_dense_kernel, act=act)
    agg_spec = (pl.BlockSpec((_NC, _TILE, D), lambda i: (0, i, 0))
                if agg.ndim == 3 else pl.BlockSpec((_TILE, D), lambda i: (i, 0)))
    return pl.pallas_call(
        kern,
        grid=(grid,),
        in_specs=[
            agg_spec,
            pl.BlockSpec((_TILE, D), lambda i: (i, 0)),
            pl.BlockSpec((_TILE, 1), lambda i: (i, 0)),
            pl.BlockSpec((NUM_GRAPHS, 1), lambda i: (0, 0)),
            pl.BlockSpec((D, 2 * D), lambda i: (0, 0)),
            pl.BlockSpec((1, 2 * D), lambda i: (0, 0)),
            pl.BlockSpec((2 * D, D), lambda i: (0, 0)),
            pl.BlockSpec((1, D), lambda i: (0, 0)),
            pl.BlockSpec((1, D), lambda i: (0, 0)),
            pl.BlockSpec((1, D), lambda i: (0, 0)),
        ],
        out_specs=pl.BlockSpec((_TILE, D), lambda i: (i, 0)),
        out_shape=jax.ShapeDtypeStruct((n_rows, D), jnp.float32),
        interpret=_INTERPRET,
    )(agg, resid, batch, invs_pg,
      bp["W1"], bp["b1"].reshape(1, 2 * D), bp["W2"], bp["b2"].reshape(1, D),
      bp["ln_g"].reshape(1, D), bp["ln_b"].reshape(1, D))


def _graph_repr_kernel(node_ref, batch_ref, counts_ref, out_ref):
    i = pl.program_id(0)
    n = pl.num_programs(0)

    @pl.when(i == 0)
    def _init():
        out_ref[...] = jnp.zeros_like(out_ref)

    b = batch_ref[...].reshape(1, _TILE)
    iota = jax.lax.broadcasted_iota(jnp.int32, (NUM_GRAPHS, 1), 0)
    p = (iota == b).astype(jnp.float32)
    out_ref[...] += jnp.dot(p, node_ref[...],
                            preferred_element_type=jnp.float32)

    @pl.when(i == n - 1)
    def _fin():
        out_ref[...] = out_ref[...] / jnp.maximum(counts_ref[...], 1.0)


def _graph_repr_call(node_pad, batch_pad, counts_pg):
    grid = _ATOM_PAD // _TILE
    return pl.pallas_call(
        _graph_repr_kernel,
        grid=(grid,),
        in_specs=[
            pl.BlockSpec((_TILE, D), lambda i: (i, 0)),
            pl.BlockSpec((_TILE, 1), lambda i: (i, 0)),
            pl.BlockSpec((NUM_GRAPHS, 1), lambda i: (0, 0)),
        ],
        out_specs=pl.BlockSpec((NUM_GRAPHS, D), lambda i: (0, 0)),
        out_shape=jax.ShapeDtypeStruct((NUM_GRAPHS, D), jnp.float32),
        interpret=_INTERPRET,
    )(node_pad, batch_pad, counts_pg)


def _gather_segsum(node_hidden, edge_hidden, edges, num_nodes):
    src, dst = edges[0], edges[1]
    msg = jnp.take(node_hidden, src, axis=0) + edge_hidden
    return jax.ops.segment_sum(msg, dst, num_segments=num_nodes)


_LSZCAP = 323840
_PAD_BASE_CH = 2
_SCAT_CH = _EDGE_PAD // _NW // 128


def _count_kernel(dst_ref, out_ref, *, shift):
    i = pl.program_id(0)

    @pl.when(i == 0)
    def _():
        out_ref[...] = jnp.zeros_like(out_ref)

    bkt = lax.shift_right_logical(dst_ref[...], shift)
    iota = jax.lax.broadcasted_iota(jnp.int32, (1, 128), 1)
    oh = (bkt == iota).astype(jnp.float32)
    out_ref[...] += jnp.sum(oh, axis=0, keepdims=True)


def _count_call(dst_col, shift):
    kern = functools.partial(_count_kernel, shift=shift)
    return pl.pallas_call(
        kern,
        grid=(_EDGE_PAD // _TILE,),
        in_specs=[pl.BlockSpec((_TILE, 1), lambda i: (i, 0))],
        out_specs=pl.BlockSpec((8, 128), lambda i: (0, 0)),
        out_shape=jax.ShapeDtypeStruct((8, 128), jnp.float32),
        interpret=_INTERPRET,
    )(dst_col)


def _pos_kernel(dst_ref, bases_ref, pos_ref, run_ref, *, shift):
    i = pl.program_id(0)

    @pl.when(i == 0)
    def _():
        run_ref[...] = jnp.zeros_like(run_ref)

    bkt = lax.shift_right_logical(dst_ref[...], shift)
    iota = jax.lax.broadcasted_iota(jnp.int32, (1, 128), 1)
    oh = (bkt == iota).astype(jnp.float32)
    r = jax.lax.broadcasted_iota(jnp.int32, (_TILE, _TILE), 0)
    cc = jax.lax.broadcasted_iota(jnp.int32, (_TILE, _TILE), 1)
    ltri = (r > cc).astype(jnp.float32)
    prior = jnp.dot(ltri, oh, preferred_element_type=jnp.float32)
    br = bases_ref[0:1, :].astype(jnp.float32) + run_ref[0:1, :]
    pos = jnp.sum(oh * (prior + br), axis=1, keepdims=True)
    pos_ref[...] = pos.astype(jnp.int32)
    run_ref[...] += jnp.sum(oh, axis=0, keepdims=True)


def _pos_call(dst_col, bases, shift):
    kern = functools.partial(_pos_kernel, shift=shift)
    return pl.pallas_call(
        kern,
        grid=(_EDGE_PAD // _TILE,),
        in_specs=[pl.BlockSpec((_TILE, 1), lambda i: (i, 0)),
                  pl.BlockSpec((8, 128), lambda i: (0, 0))],
        out_specs=pl.BlockSpec((_TILE, 1), lambda i: (i, 0)),
        out_shape=jax.ShapeDtypeStruct((_EDGE_PAD, 1), jnp.int32),
        scratch_shapes=[pltpu.VMEM((8, 128), jnp.float32)],
        interpret=_INTERPRET,
    )(dst_col, bases)


def _sc_permute_body(pos_hbm, src_hbm, dst_hbm, padv_hbm, perm_hbm,
                     lsrc_hbm, lldst_hbm, posv, sbuf, dbuf, lbuf, ebuf,
                     patv, pv16, padidx, s1, s2, s3, *, slab):
    c = lax.axis_index("c")
    s = lax.axis_index("s")
    wid = s * _NC + c
    iota = lax.iota(jnp.int32, 16)

    @pl.when(wid == 0)
    def _():
        pltpu.sync_copy(padv_hbm, pv16)
        pb = pv16[pl.ds(0, 16)]
        for t in range(2):
            for k in range(8):
                padidx[pl.ds(k * 16, 16)] = pb + t * 128 + k * 16 + iota
                patv[pl.ds(k * 16, 16)] = jnp.full((16,), -1, jnp.int32)
            a1 = pltpu.async_copy(patv, perm_hbm.at[padidx], s1)
            a1.wait()
            for k in range(8):
                patv[pl.ds(k * 16, 16)] = ((iota + k * 16) * 13) & 8191
            a2 = pltpu.async_copy(patv, lsrc_hbm.at[padidx], s2)
            a2.wait()
            for k in range(8):
                patv[pl.ds(k * 16, 16)] = slab + k * 16 + iota
            a3 = pltpu.async_copy(patv, lldst_hbm.at[padidx], s3)
            a3.wait()

    def scat(t, carry):
        st = wid * (_SCAT_CH * 128) + t * 128
        pltpu.sync_copy(pos_hbm.at[pl.ds(st, 128)], posv)
        pltpu.sync_copy(src_hbm.at[pl.ds(st, 128)], sbuf)
        pltpu.sync_copy(dst_hbm.at[pl.ds(st, 128)], dbuf)
        for k in range(8):
            ebuf[pl.ds(k * 16, 16)] = st + k * 16 + iota
            dv = dbuf[pl.ds(k * 16, 16)]
            lbuf[pl.ds(k * 16, 16)] = dv & (slab - 1)
        a1 = pltpu.async_copy(ebuf, perm_hbm.at[posv], s1)
        a2 = pltpu.async_copy(sbuf, lsrc_hbm.at[posv], s2)
        a3 = pltpu.async_copy(lbuf, lldst_hbm.at[posv], s3)
        a1.wait()
        a2.wait()
        a3.wait()
        return carry

    lax.fori_loop(0, _SCAT_CH, scat, 0)


def _sc_permute(pos_flat, src_flat, dst_flat, padv, slab):
    body = functools.partial(_sc_permute_body, slab=slab)
    f = pl.kernel(
        body,
        out_type=[jax.ShapeDtypeStruct((_LSZCAP,), jnp.int32)] * 3,
        mesh=plsc.VectorSubcoreMesh(core_axis_name="c", subcore_axis_name="s"),
        scratch_types=[pltpu.VMEM((128,), jnp.int32)] * 6 +
                      [pltpu.VMEM((16,), jnp.int32),
                       pltpu.VMEM((128,), jnp.int32)] + [
            pltpu.SemaphoreType.DMA,
            pltpu.SemaphoreType.DMA,
            pltpu.SemaphoreType.DMA,
        ],
    )
    return f(pos_flat, src_flat, dst_flat, padv)


def _sc_consume_body(node_hbm, edge_hbm, lsrc_hbm, perm_hbm, lldst_hbm,
                     obs_hbm, obm_hbm, zeros_hbm, out_hbm,
                     osv, omv, idxv, permv, eidv, srcv, ldstv_a, ldstv_b,
                     nbuf, ebuf, zbuf, accum,
                     s1, s2, s3, s4, s5, *, slab, npasses):
    c = lax.axis_index("c")
    s = lax.axis_index("s")
    pltpu.sync_copy(zeros_hbm, zbuf)
    rpt = slab // _NS
    iota = lax.iota(jnp.int32, 16)

    def pass_body(p, carry):
        b = p * _NC + c
        pltpu.sync_copy(obs_hbm.at[p, c], osv)
        pltpu.sync_copy(obm_hbm.at[p, c], omv)
        sv = osv[pl.ds(0, 16)]
        mv = omv[pl.ds(0, 16)]
        nch = mv[0]
        valid = mv[1]
        for k in range(rpt // 128):
            pltpu.sync_copy(zbuf, accum.at[pl.ds(s * rpt + k * 128, 128)])
        plsc.subcore_barrier()

        seglen = mv[2]
        pb = mv[3]

        def chunk(jj, carry2):
            ch = s + jj * _NS
            for k in range(16):
                lane = ch * 256 + k * 16 + iota
                idxv[pl.ds(k * 16, 16)] = jnp.where(
                    lane < seglen, sv + lane, pb + (lane & 127))
            a1 = pltpu.async_copy(perm_hbm.at[idxv], permv, s1)
            a2 = pltpu.async_copy(lsrc_hbm.at[idxv], srcv, s2)
            a3 = pltpu.async_copy(
                lldst_hbm.at[idxv.at[pl.ds(0, 128)]], ldstv_a, s3)
            a4 = pltpu.async_copy(
                lldst_hbm.at[idxv.at[pl.ds(128, 128)]], ldstv_b, s3)
            a1.wait()
            for k in range(16):
                pv = permv[pl.ds(k * 16, 16)]
                eidv[pl.ds(k * 16, 16)] = jnp.where(
                    pv < 0, ((iota + k * 16) * 13) & 8191, pv)
            a2.wait()
            a3.wait()
            a4.wait()
            g1 = pltpu.async_copy(
                node_hbm.at[srcv.at[pl.ds(0, 128)]], nbuf, s4)
            g2 = pltpu.async_copy(
                node_hbm.at[srcv.at[pl.ds(128, 128)]], ebuf, s5)
            g1.wait()
            g3 = pltpu.async_copy(
                edge_hbm.at[eidv.at[pl.ds(0, 128)]], nbuf, s4, add=True)
            g2.wait()
            g4 = pltpu.async_copy(
                edge_hbm.at[eidv.at[pl.ds(128, 128)]], ebuf, s5, add=True)
            g3.wait()
            w1 = pltpu.async_copy(nbuf, accum.at[ldstv_a], s4, add=True)
            g4.wait()
            w2 = pltpu.async_copy(ebuf, accum.at[ldstv_b], s5, add=True)
            w1.wait()
            w2.wait()
            return carry2

        ntr = jnp.maximum((nch - s + _NS - 1) // _NS, 0)
        lax.fori_loop(0, ntr, chunk, 0)
        plsc.subcore_barrier()

        @pl.when(valid > 0)
        def _():
            pltpu.sync_copy(
                accum.at[pl.ds(s * rpt, rpt)],
                out_hbm.at[pl.ds(b * slab + s * rpt, rpt)])

        return carry

    lax.fori_loop(0, npasses, pass_body, 0)


def _sc_consume(node_pad, edge_pad, lsrc, perm, lldst, obs, obm, nb, slab):
    npasses = (nb + _NC - 1) // _NC
    body = functools.partial(_sc_consume_body, slab=slab, npasses=npasses)
    f = pl.kernel(
        body,
        out_type=jax.ShapeDtypeStruct((nb * slab, D), jnp.float32),
        mesh=plsc.VectorSubcoreMesh(core_axis_name="c", subcore_axis_name="s"),
        scratch_types=[
            pltpu.VMEM((16,), jnp.int32),
            pltpu.VMEM((16,), jnp.int32),
            pltpu.VMEM((256,), jnp.int32),
            pltpu.VMEM((256,), jnp.int32),
            pltpu.VMEM((256,), jnp.int32),
            pltpu.VMEM((256,), jnp.int32),
            pltpu.VMEM((128,), jnp.int32),
            pltpu.VMEM((128,), jnp.int32),
            pltpu.VMEM((_ECHUNK, D), jnp.float32),
            pltpu.VMEM((_ECHUNK, D), jnp.float32),
            pltpu.VMEM((_ECHUNK, D), jnp.float32),
            pltpu.VMEM_SHARED((slab + _ECHUNK, D), jnp.float32),
            pltpu.SemaphoreType.DMA,
            pltpu.SemaphoreType.DMA,
            pltpu.SemaphoreType.DMA,
            pltpu.SemaphoreType.DMA,
            pltpu.SemaphoreType.DMA,
        ],
    )
    zeros128 = jnp.zeros((_ECHUNK, D), jnp.float32)
    return f(node_pad, edge_pad, lsrc, perm, lldst, obs, obm, zeros128)


def _graph_lists(src, dst, n_nodes, nb, shift):
    slab = 1 << shift
    npad = _EDGE_PAD - src.shape[0]
    ar = jnp.arange(npad, dtype=jnp.int32)
    src_f = jnp.concatenate([src.astype(jnp.int32), ar % n_nodes])
    dst_f = jnp.concatenate([dst.astype(jnp.int32),
                             n_nodes + ar % (nb * slab - n_nodes)])
    counts = _count_call(dst_f.reshape(-1, 1), shift)[0].astype(jnp.int32)
    off = jnp.concatenate([jnp.zeros((1,), jnp.int32), jnp.cumsum(counts)])
    bases = jnp.zeros((8, 128), jnp.int32).at[0, :65].set(off[:65])
    pos = _pos_call(dst_f.reshape(-1, 1), bases, shift)
    padbase = off[nb]
    padv = jnp.broadcast_to(padbase.reshape(1), (16,)).astype(jnp.int32)
    perm, lsrc, lldst = _sc_permute(pos.reshape(-1), src_f, dst_f, padv, slab)
    npasses = (nb + _NC - 1) // _NC
    nch = (off[1:] - off[:-1] + 255) // 256
    starts = off[:nb]
    nchb = nch[:nb]
    segl = (off[1:] - off[:-1])[:nb]
    validb = jnp.ones((nb,), jnp.int32)
    if nb % _NC:
        starts = jnp.concatenate([starts, jnp.zeros((1,), jnp.int32)])
        nchb = jnp.concatenate([nchb, jnp.zeros((1,), jnp.int32)])
        segl = jnp.concatenate([segl, jnp.zeros((1,), jnp.int32)])
        validb = jnp.concatenate([validb, jnp.zeros((1,), jnp.int32)])
    obs = jnp.zeros((32, _NC, 16), jnp.int32)
    obs = obs.at[:npasses].set(
        jnp.broadcast_to(starts.reshape(npasses, _NC, 1), (npasses, _NC, 16)))
    obm = jnp.zeros((32, _NC, 16), jnp.int32)
    obm = obm.at[:npasses, :, 0].set(nchb.reshape(npasses, _NC))
    obm = obm.at[:npasses, :, 1].set(validb.reshape(npasses, _NC))
    obm = obm.at[:npasses, :, 2].set(segl.reshape(npasses, _NC))
    obm = obm.at[:npasses, :, 3].set(jnp.broadcast_to(
        padbase.reshape(1, 1), (npasses, _NC)).astype(jnp.int32))
    return lsrc, perm, lldst, obs, obm


def kernel(AtomBondGraph_edges, BondAngleGraph_edges, AngleDihedralGraph_edges,
           x, bond_attr, bond_lengths, bond_angles, dihedral_angles,
           atom_batch, num_bonds, num_angles, num_graphs,
           masked_atom_indices, masked_bond_indices, masked_angle_indices,
           masked_dihedral_indices, params):
    _x = x.at[masked_atom_indices].set(15)
    _battr = bond_attr.at[masked_bond_indices].set(7)
    _bl = bond_lengths.at[masked_bond_indices].set(0.0)
    _ang = bond_angles.at[masked_angle_indices].set(0.0)
    _dih = dihedral_angles.at[masked_dihedral_indices].set(0.0)

    sb = jnp.searchsorted(atom_batch, jnp.arange(NUM_GRAPHS + 1, dtype=atom_batch.dtype))
    atom_counts = (sb[1:] - sb[:-1]).astype(jnp.float32)
    inv_atoms = jax.lax.rsqrt(jnp.maximum(atom_counts, 1.0)).reshape(NUM_GRAPHS, 1)
    bond_counts = num_bonds.astype(jnp.float32)
    inv_bonds = jax.lax.rsqrt(jnp.maximum(bond_counts, 1.0)).reshape(NUM_GRAPHS, 1)
    angle_counts = num_angles.astype(jnp.float32)
    inv_angles = jax.lax.rsqrt(jnp.maximum(angle_counts, 1.0)).reshape(NUM_GRAPHS, 1)

    gid = jnp.arange(NUM_GRAPHS, dtype=jnp.int32)
    bond_batch = jnp.repeat(gid, num_bonds, total_repeat_length=N_BONDS)
    angle_batch = jnp.repeat(gid, num_angles, total_repeat_length=N_ANGLES)
    ab_pad = _pad_rows(atom_batch.astype(jnp.int32).reshape(-1, 1), _ATOM_PAD, -1)
    bb_pad = _pad_rows(bond_batch.reshape(-1, 1), _EDGE_PAD, -1)
    anb_pad = _pad_rows(angle_batch.reshape(-1, 1), _EDGE_PAD, -1)

    atom_tab = jnp.concatenate(params["atom_emb"], axis=0)
    x_pad = _pad_rows(_x.astype(jnp.int32), _ATOM_PAD)
    node_hidden = _embed_call(x_pad, atom_tab, 16, 7, None, _ATOM_PAD)[:N_ATOMS]

    def bond_feat_input():
        battr_pad = _pad_rows(_battr.astype(jnp.int32), _EDGE_PAD)
        bl_bits = jax.lax.bitcast_convert_type(
            _bl.astype(jnp.float32), jnp.int32).reshape(-1, 1)
        return jnp.concatenate([battr_pad, _pad_rows(bl_bits, _EDGE_PAD)], axis=1)

    bond_feats = bond_feat_input()

    def bond_embed(tables, rbf_p):
        tab = jnp.concatenate(list(tables) + [rbf_p["W"]], axis=0)
        out = _embed_call(bond_feats, tab, 8, 3, _BL_CENTERS, _EDGE_PAD)
        return out + rbf_p["b"][None, :]

    bond_hidden = bond_embed(params["init_bond_emb"], params["init_bond_rbf"])[:N_BONDS]

    ang_pad = _pad_rows(_ang.astype(jnp.float32), _EDGE_PAD)
    dih_pad = _pad_rows(_dih.astype(jnp.float32), _EDGE_PAD)
    angle_hidden = _rbf_call(ang_pad, params["init_angle_rbf"]["W"],
                             params["init_angle_rbf"]["b"], _BA_CENTERS,
                             _EDGE_PAD)[:N_ANGLES]

    ab_g = _graph_lists(AtomBondGraph_edges[0], AtomBondGraph_edges[1],
                        N_ATOMS, 3, 12)
    ba_g = _graph_lists(BondAngleGraph_edges[0], BondAngleGraph_edges[1],
                        N_BONDS, 40, 13)
    ad_g = _graph_lists(AngleDihedralGraph_edges[0], AngleDihedralGraph_edges[1],
                        N_ANGLES, 40, 13)

    nh_pad = _pad_rows(node_hidden, _ATOM_PAD)
    eh_pad = _pad_rows(bond_hidden, _EDGE_PAD)
    ah_pad = _pad_rows(angle_hidden, _EDGE_PAD)
    dih_hidden = None
    for lid in range(N_LAYERS):
        lp = params["layers"][lid]
        act = lid != N_LAYERS - 1

        agg_a = _sc_consume(nh_pad, eh_pad, ab_g[0], ab_g[1], ab_g[2],
                            ab_g[3], ab_g[4], 3, 4096)
        nh_pad = _block_dense_call(
            agg_a, nh_pad, ab_pad, inv_atoms, lp["ab_block"], act, _ATOM_PAD)

        cur_edge_pad = bond_embed(lp["bond_emb"], lp["bond_rbf"])
        agg_b = _sc_consume(cur_edge_pad, ah_pad, ba_g[0], ba_g[1], ba_g[2],
                            ba_g[3], ba_g[4], 40, 8192)
        eh_pad = _block_dense_call(
            agg_b, cur_edge_pad, bb_pad,
            inv_bonds, lp["ba_block"], act, _EDGE_PAD)

        cur_angle_pad = _rbf_call(ang_pad, lp["angle_rbf"]["W"],
                                  lp["angle_rbf"]["b"], _BA_CENTERS, _EDGE_PAD)
        dih_hidden = _rbf_call(dih_pad, lp["dihedral_rbf"]["W"],
                               lp["dihedral_rbf"]["b"], _DA_CENTERS, _EDGE_PAD)
        agg_an = _sc_consume(cur_angle_pad, dih_hidden, ad_g[0], ad_g[1],
                             ad_g[2], ad_g[3], ad_g[4], 40, 8192)
        ah_pad = _block_dense_call(
            agg_an, cur_angle_pad,
            anb_pad, inv_angles, lp["ad_block"], act, _EDGE_PAD)

    graph_repr = _graph_repr_call(nh_pad, ab_pad,
                                  atom_counts.reshape(NUM_GRAPHS, 1))
    return (nh_pad[:N_ATOMS], eh_pad[:N_BONDS], ah_pad[:N_ANGLES],
            dih_hidden[:N_DIHEDRALS], graph_repr)


# speedup vs baseline: 1.0774x; 1.0721x over previous
import functools

import jax
import jax.numpy as jnp
import numpy as np
from jax import lax
from jax.experimental import pallas as pl
from jax.experimental.pallas import tpu as pltpu
from jax.experimental.pallas import tpu_sc as plsc

_INTERPRET = False

D = 128
N_ATOMS = 10000
N_BONDS = 319600
N_ANGLES = 319600
N_DIHEDRALS = 319600
NUM_GRAPHS = 800
N_LAYERS = 3
GAMMA = 10.0
_BL_CENTERS = np.arange(0.0, 2.0, 0.1).astype(np.float32)
_BA_CENTERS = np.arange(0.0, np.pi, 0.1).astype(np.float32)
_DA_CENTERS = np.arange(-np.pi, np.pi, 0.2).astype(np.float32)

_TILE = 512
_ATOM_PAD = 10240
_EDGE_PAD = 323584
_NC = 2
_NS = 16
_NW = _NC * _NS
_EPW = _EDGE_PAD // _NW
_ECHUNK = 128
_NCHUNKS = _EPW // _ECHUNK


def _pad_rows(a, n, value=0):
    return jnp.pad(a, ((0, n - a.shape[0]),) + ((0, 0),) * (a.ndim - 1),
                   constant_values=value)


def _embed_kernel(feats_ref, table_ref, centers_ref, out_ref, *, vocab, ncols):
    f = feats_ref[...]
    iota = jax.lax.broadcasted_iota(jnp.int32, (1, vocab), 1)
    blocks = [(f[:, j:j + 1] == iota).astype(jnp.float32) for j in range(ncols)]
    if centers_ref is not None:
        xs = jax.lax.bitcast_convert_type(f[:, ncols:ncols + 1], jnp.float32)
        blocks.append(jnp.exp(-GAMMA * (xs - centers_ref[...]) ** 2))
    oh = jnp.concatenate(blocks, axis=1)
    out_ref[...] = jnp.dot(oh, table_ref[...],
                           preferred_element_type=jnp.float32)


def _embed_call(feats_f32col, tables_stacked, vocab, ncols, centers, n_rows):
    grid = n_rows // _TILE
    has_c = centers is not None
    if has_c:
        kern = functools.partial(_embed_kernel, vocab=vocab, ncols=ncols)
    else:
        kern = functools.partial(
            lambda fr, tr, outr, **kw: _embed_kernel(fr, tr, None, outr, **kw),
            vocab=vocab, ncols=ncols)
    in_specs = [
        pl.BlockSpec((_TILE, feats_f32col.shape[1]), lambda i: (i, 0)),
        pl.BlockSpec(tables_stacked.shape, lambda i: (0, 0)),
    ]
    args = [feats_f32col, tables_stacked]
    if has_c:
        c = jnp.asarray(centers).reshape(1, -1)
        in_specs.append(pl.BlockSpec(c.shape, lambda i: (0, 0)))
        args.append(c)
    return pl.pallas_call(
        kern,
        grid=(grid,),
        in_specs=in_specs,
        out_specs=pl.BlockSpec((_TILE, D), lambda i: (i, 0)),
        out_shape=jax.ShapeDtypeStruct((n_rows, D), jnp.float32),
        interpret=_INTERPRET,
    )(*args)


def _rbf_kernel(x_ref, w_ref, b_ref, c_ref, out_ref):
    x = x_ref[...]
    feats = jnp.exp(-GAMMA * (x - c_ref[...]) ** 2)
    out_ref[...] = jnp.dot(feats, w_ref[...],
                           preferred_element_type=jnp.float32) + b_ref[...]


def _rbf_call(x, w, b, centers, n_rows):
    grid = n_rows // _TILE
    c = jnp.asarray(centers).reshape(1, -1)
    return pl.pallas_call(
        _rbf_kernel,
        grid=(grid,),
        in_specs=[
            pl.BlockSpec((_TILE, 1), lambda i: (i, 0)),
            pl.BlockSpec(w.shape, lambda i: (0, 0)),
            pl.BlockSpec((1, D), lambda i: (0, 0)),
            pl.BlockSpec(c.shape, lambda i: (0, 0)),
        ],
        out_specs=pl.BlockSpec((_TILE, D), lambda i: (i, 0)),
        out_shape=jax.ShapeDtypeStruct((n_rows, D), jnp.float32),
        interpret=_INTERPRET,
    )(x, w, b.reshape(1, D), c)


def _block_dense_kernel(agg_ref, resid_ref, batch_ref, invs_ref,
                        w1_ref, b1_ref, w2_ref, b2_ref, g_ref, bb_ref,
                        out_ref, *, act):
    agg = agg_ref[...]
    if agg.ndim == 3:
        agg = agg[0] + agg[1]
    h = jnp.dot(agg, w1_ref[...], preferred_element_type=jnp.float32) + b1_ref[...]
    h = jnp.maximum(h, 0.0)
    h = jnp.dot(h, w2_ref[...], preferred_element_type=jnp.float32) + b2_ref[...]
    mu = jnp.mean(h, axis=-1, keepdims=True)
    var = jnp.mean((h - mu) ** 2, axis=-1, keepdims=True)
    h = (h - mu) * jax.lax.rsqrt(var + 1e-5) * g_ref[...] + bb_ref[...]
    b = batch_ref[...]
    iota = jax.lax.broadcasted_iota(jnp.int32, (1, NUM_GRAPHS), 1)
    onehot = (b == iota).astype(jnp.float32)
    scale = jnp.dot(onehot, invs_ref[...],
                    preferred_element_type=jnp.float32)
    h = h * scale
    if act:
        h = jnp.maximum(h, 0.0)
    out_ref[...] = h + resid_ref[...]


def _block_dense_call(agg, resid, batch, invs_pg, bp, act, n_rows):
    grid = n_rows // _TILE
    kern = functools.partial(_block_dense_kernel, act=act)
    agg_spec = (pl.BlockSpec((_NC, _TILE, D), lambda i: (0, i, 0))
                if agg.ndim == 3 else pl.BlockSpec((_TILE, D), lambda i: (i, 0)))
    return pl.pallas_call(
        kern,
        grid=(grid,),
        in_specs=[
            agg_spec,
            pl.BlockSpec((_TILE, D), lambda i: (i, 0)),
            pl.BlockSpec((_TILE, 1), lambda i: (i, 0)),
            pl.BlockSpec((NUM_GRAPHS, 1), lambda i: (0, 0)),
            pl.BlockSpec((D, 2 * D), lambda i: (0, 0)),
            pl.BlockSpec((1, 2 * D), lambda i: (0, 0)),
            pl.BlockSpec((2 * D, D), lambda i: (0, 0)),
            pl.BlockSpec((1, D), lambda i: (0, 0)),
            pl.BlockSpec((1, D), lambda i: (0, 0)),
            pl.BlockSpec((1, D), lambda i: (0, 0)),
        ],
        out_specs=pl.BlockSpec((_TILE, D), lambda i: (i, 0)),
        out_shape=jax.ShapeDtypeStruct((n_rows, D), jnp.float32),
        interpret=_INTERPRET,
    )(agg, resid, batch, invs_pg,
      bp["W1"], bp["b1"].reshape(1, 2 * D), bp["W2"], bp["b2"].reshape(1, D),
      bp["ln_g"].reshape(1, D), bp["ln_b"].reshape(1, D))


def _graph_repr_kernel(node_ref, batch_ref, counts_ref, out_ref):
    i = pl.program_id(0)
    n = pl.num_programs(0)

    @pl.when(i == 0)
    def _init():
        out_ref[...] = jnp.zeros_like(out_ref)

    b = batch_ref[...].reshape(1, _TILE)
    iota = jax.lax.broadcasted_iota(jnp.int32, (NUM_GRAPHS, 1), 0)
    p = (iota == b).astype(jnp.float32)
    out_ref[...] += jnp.dot(p, node_ref[...],
                            preferred_element_type=jnp.float32)

    @pl.when(i == n - 1)
    def _fin():
        out_ref[...] = out_ref[...] / jnp.maximum(counts_ref[...], 1.0)


def _graph_repr_call(node_pad, batch_pad, counts_pg):
    grid = _ATOM_PAD // _TILE
    return pl.pallas_call(
        _graph_repr_kernel,
        grid=(grid,),
        in_specs=[
            pl.BlockSpec((_TILE, D), lambda i: (i, 0)),
            pl.BlockSpec((_TILE, 1), lambda i: (i, 0)),
            pl.BlockSpec((NUM_GRAPHS, 1), lambda i: (0, 0)),
        ],
        out_specs=pl.BlockSpec((NUM_GRAPHS, D), lambda i: (0, 0)),
        out_shape=jax.ShapeDtypeStruct((NUM_GRAPHS, D), jnp.float32),
        interpret=_INTERPRET,
    )(node_pad, batch_pad, counts_pg)


def _gather_segsum(node_hidden, edge_hidden, edges, num_nodes):
    src, dst = edges[0], edges[1]
    msg = jnp.take(node_hidden, src, axis=0) + edge_hidden
    return jax.ops.segment_sum(msg, dst, num_segments=num_nodes)


_LSZCAP = 323840
_PAD_BASE_CH = 2
_SCAT_CH = _EDGE_PAD // _NW // 128


def _count_kernel(dst_ref, out_ref, *, shift):
    i = pl.program_id(0)

    @pl.when(i == 0)
    def _():
        out_ref[...] = jnp.zeros_like(out_ref)

    bkt = lax.shift_right_logical(dst_ref[...], shift)
    iota = jax.lax.broadcasted_iota(jnp.int32, (1, 128), 1)
    oh = (bkt == iota).astype(jnp.float32)
    out_ref[...] += jnp.sum(oh, axis=0, keepdims=True)


def _count_call(dst_col, shift):
    kern = functools.partial(_count_kernel, shift=shift)
    return pl.pallas_call(
        kern,
        grid=(_EDGE_PAD // _TILE,),
        in_specs=[pl.BlockSpec((_TILE, 1), lambda i: (i, 0))],
        out_specs=pl.BlockSpec((8, 128), lambda i: (0, 0)),
        out_shape=jax.ShapeDtypeStruct((8, 128), jnp.float32),
        interpret=_INTERPRET,
    )(dst_col)


def _pos_kernel(dst_ref, bases_ref, pos_ref, run_ref, *, shift):
    i = pl.program_id(0)

    @pl.when(i == 0)
    def _():
        run_ref[...] = jnp.zeros_like(run_ref)

    bkt = lax.shift_right_logical(dst_ref[...], shift)
    iota = jax.lax.broadcasted_iota(jnp.int32, (1, 128), 1)
    oh = (bkt == iota).astype(jnp.float32)
    r = jax.lax.broadcasted_iota(jnp.int32, (_TILE, _TILE), 0)
    cc = jax.lax.broadcasted_iota(jnp.int32, (_TILE, _TILE), 1)
    ltri = (r > cc).astype(jnp.float32)
    prior = jnp.dot(ltri, oh, preferred_element_type=jnp.float32)
    br = bases_ref[0:1, :].astype(jnp.float32) + run_ref[0:1, :]
    pos = jnp.sum(oh * (prior + br), axis=1, keepdims=True)
    pos_ref[...] = pos.astype(jnp.int32)
    run_ref[...] += jnp.sum(oh, axis=0, keepdims=True)


def _pos_call(dst_col, bases, shift):
    kern = functools.partial(_pos_kernel, shift=shift)
    return pl.pallas_call(
        kern,
        grid=(_EDGE_PAD // _TILE,),
        in_specs=[pl.BlockSpec((_TILE, 1), lambda i: (i, 0)),
                  pl.BlockSpec((8, 128), lambda i: (0, 0))],
        out_specs=pl.BlockSpec((_TILE, 1), lambda i: (i, 0)),
        out_shape=jax.ShapeDtypeStruct((_EDGE_PAD, 1), jnp.int32),
        scratch_shapes=[pltpu.VMEM((8, 128), jnp.float32)],
        interpret=_INTERPRET,
    )(dst_col, bases)


def _sc_permute_body(pos_hbm, src_hbm, dst_hbm, padv_hbm, perm_hbm,
                     lsrc_hbm, lldst_hbm, posv, sbuf, dbuf, lbuf, ebuf,
                     patv, pv16, padidx, s1, s2, s3, *, slab):
    c = lax.axis_index("c")
    s = lax.axis_index("s")
    wid = s * _NC + c
    iota = lax.iota(jnp.int32, 16)

    @pl.when(wid == 0)
    def _():
        pltpu.sync_copy(padv_hbm, pv16)
        pb = pv16[pl.ds(0, 16)]
        for t in range(2):
            for k in range(8):
                padidx[pl.ds(k * 16, 16)] = pb + t * 128 + k * 16 + iota
                patv[pl.ds(k * 16, 16)] = jnp.full((16,), -1, jnp.int32)
            a1 = pltpu.async_copy(patv, perm_hbm.at[padidx], s1)
            a1.wait()
            for k in range(8):
                patv[pl.ds(k * 16, 16)] = ((iota + k * 16) * 13) & 8191
            a2 = pltpu.async_copy(patv, lsrc_hbm.at[padidx], s2)
            a2.wait()
            for k in range(8):
                patv[pl.ds(k * 16, 16)] = slab + k * 16 + iota
            a3 = pltpu.async_copy(patv, lldst_hbm.at[padidx], s3)
            a3.wait()

    def scat(t, carry):
        st = wid * (_SCAT_CH * 128) + t * 128
        pltpu.sync_copy(pos_hbm.at[pl.ds(st, 128)], posv)
        pltpu.sync_copy(src_hbm.at[pl.ds(st, 128)], sbuf)
        pltpu.sync_copy(dst_hbm.at[pl.ds(st, 128)], dbuf)
        for k in range(8):
            ebuf[pl.ds(k * 16, 16)] = st + k * 16 + iota
            dv = dbuf[pl.ds(k * 16, 16)]
            lbuf[pl.ds(k * 16, 16)] = dv & (slab - 1)
        a1 = pltpu.async_copy(ebuf, perm_hbm.at[posv], s1)
        a2 = pltpu.async_copy(sbuf, lsrc_hbm.at[posv], s2)
        a3 = pltpu.async_copy(lbuf, lldst_hbm.at[posv], s3)
        a1.wait()
        a2.wait()
        a3.wait()
        return carry

    lax.fori_loop(0, _SCAT_CH, scat, 0)


def _sc_permute(pos_flat, src_flat, dst_flat, padv, slab):
    body = functools.partial(_sc_permute_body, slab=slab)
    f = pl.kernel(
        body,
        out_type=[jax.ShapeDtypeStruct((_LSZCAP,), jnp.int32)] * 3,
        mesh=plsc.VectorSubcoreMesh(core_axis_name="c", subcore_axis_name="s"),
        scratch_types=[pltpu.VMEM((128,), jnp.int32)] * 6 +
                      [pltpu.VMEM((16,), jnp.int32),
                       pltpu.VMEM((128,), jnp.int32)] + [
            pltpu.SemaphoreType.DMA,
            pltpu.SemaphoreType.DMA,
            pltpu.SemaphoreType.DMA,
        ],
    )
    return f(pos_flat, src_flat, dst_flat, padv)


def _sc_atoms_body(node_hbm, edge_hbm, src_hbm, dst_hbm, zeros_hbm, out_hbm,
                   srcv, dstv, nbuf, ebuf, zbuf, accum, s1, s2):
    c = lax.axis_index("c")
    s = lax.axis_index("s")
    wid = s * _NC + c
    rpt = _ATOM_PAD // _NS
    pltpu.sync_copy(zeros_hbm, zbuf)
    for k in range(rpt // 64):
        pltpu.sync_copy(zbuf, accum.at[pl.ds(s * rpt + k * 64, 64)])
    plsc.subcore_barrier()
    epw = _EDGE_PAD // _NW

    def chunk(j, carry):
        base = wid * epw + j * 128
        pltpu.sync_copy(src_hbm.at[pl.ds(base, 128)], srcv)
        pltpu.sync_copy(dst_hbm.at[pl.ds(base, 128)], dstv)
        g1 = pltpu.async_copy(node_hbm.at[srcv], nbuf, s1)
        g2 = pltpu.async_copy(edge_hbm.at[pl.ds(base, 128)], ebuf, s2)
        g1.wait()
        w1 = pltpu.async_copy(nbuf, accum.at[dstv], s1, add=True)
        g2.wait()
        w2 = pltpu.async_copy(ebuf, accum.at[dstv], s2, add=True)
        w1.wait()
        w2.wait()
        return carry

    lax.fori_loop(0, epw // 128, chunk, 0)
    plsc.subcore_barrier()
    pltpu.sync_copy(accum.at[pl.ds(s * rpt, rpt)],
                    out_hbm.at[c, pl.ds(s * rpt, rpt)])


def _sc_atoms(node_pad, edge_pad, src_flat, dst_flat):
    f = pl.kernel(
        _sc_atoms_body,
        out_type=jax.ShapeDtypeStruct((_NC, _ATOM_PAD, D), jnp.float32),
        mesh=plsc.VectorSubcoreMesh(core_axis_name="c", subcore_axis_name="s"),
        scratch_types=[
            pltpu.VMEM((128,), jnp.int32),
            pltpu.VMEM((128,), jnp.int32),
            pltpu.VMEM((128, D), jnp.float32),
            pltpu.VMEM((128, D), jnp.float32),
            pltpu.VMEM((64, D), jnp.float32),
            pltpu.VMEM_SHARED((_ATOM_PAD, D), jnp.float32),
            pltpu.SemaphoreType.DMA,
            pltpu.SemaphoreType.DMA,
        ],
    )
    zeros64 = jnp.zeros((64, D), jnp.float32)
    return f(node_pad, edge_pad, src_flat, dst_flat, zeros64)


def _sc_consume_body(node_hbm, edge_hbm, lsrc_hbm, perm_hbm, lldst_hbm,
                     obs_hbm, obm_hbm, zeros_hbm, out_hbm,
                     osv, omv, idxv, permv, eidv, srcv, ldstv_a, ldstv_b,
                     nbuf, ebuf, zbuf, accum,
                     s1, s2, s3, s4, s5, *, slab, npasses):
    c = lax.axis_index("c")
    s = lax.axis_index("s")
    pltpu.sync_copy(zeros_hbm, zbuf)
    rpt = slab // _NS
    iota = lax.iota(jnp.int32, 16)

    def pass_body(p, carry):
        b = p * _NC + c
        pltpu.sync_copy(obs_hbm.at[p, c], osv)
        pltpu.sync_copy(obm_hbm.at[p, c], omv)
        sv = osv[pl.ds(0, 16)]
        mv = omv[pl.ds(0, 16)]
        nch = mv[0]
        valid = mv[1]
        for k in range(rpt // 128):
            pltpu.sync_copy(zbuf, accum.at[pl.ds(s * rpt + k * 128, 128)])
        plsc.subcore_barrier()

        seglen = mv[2]
        pb = mv[3]

        def chunk(jj, carry2):
            ch = s + jj * _NS
            for k in range(16):
                lane = ch * 256 + k * 16 + iota
                idxv[pl.ds(k * 16, 16)] = jnp.where(
                    lane < seglen, sv + lane, pb + (lane & 127))
            a1 = pltpu.async_copy(perm_hbm.at[idxv], permv, s1)
            a2 = pltpu.async_copy(lsrc_hbm.at[idxv], srcv, s2)
            a3 = pltpu.async_copy(
                lldst_hbm.at[idxv.at[pl.ds(0, 128)]], ldstv_a, s3)
            a4 = pltpu.async_copy(
                lldst_hbm.at[idxv.at[pl.ds(128, 128)]], ldstv_b, s3)
            a1.wait()
            for k in range(16):
                pv = permv[pl.ds(k * 16, 16)]
                eidv[pl.ds(k * 16, 16)] = jnp.where(
                    pv < 0, ((iota + k * 16) * 13) & 8191, pv)
            a2.wait()
            a3.wait()
            a4.wait()
            g1 = pltpu.async_copy(
                node_hbm.at[srcv.at[pl.ds(0, 128)]], nbuf, s4)
            g2 = pltpu.async_copy(
                node_hbm.at[srcv.at[pl.ds(128, 128)]], ebuf, s5)
            g1.wait()
            g3 = pltpu.async_copy(
                edge_hbm.at[eidv.at[pl.ds(0, 128)]], nbuf, s4, add=True)
            g2.wait()
            g4 = pltpu.async_copy(
                edge_hbm.at[eidv.at[pl.ds(128, 128)]], ebuf, s5, add=True)
            g3.wait()
            w1 = pltpu.async_copy(nbuf, accum.at[ldstv_a], s4, add=True)
            g4.wait()
            w2 = pltpu.async_copy(ebuf, accum.at[ldstv_b], s5, add=True)
            w1.wait()
            w2.wait()
            return carry2

        ntr = jnp.maximum((nch - s + _NS - 1) // _NS, 0)
        lax.fori_loop(0, ntr, chunk, 0)
        plsc.subcore_barrier()

        @pl.when(valid > 0)
        def _():
            pltpu.sync_copy(
                accum.at[pl.ds(s * rpt, rpt)],
                out_hbm.at[pl.ds(b * slab + s * rpt, rpt)])

        return carry

    lax.fori_loop(0, npasses, pass_body, 0)


def _sc_consume(node_pad, edge_pad, lsrc, perm, lldst, obs, obm, nb, slab):
    npasses = (nb + _NC - 1) // _NC
    body = functools.partial(_sc_consume_body, slab=slab, npasses=npasses)
    f = pl.kernel(
        body,
        out_type=jax.ShapeDtypeStruct((nb * slab, D), jnp.float32),
        mesh=plsc.VectorSubcoreMesh(core_axis_name="c", subcore_axis_name="s"),
        scratch_types=[
            pltpu.VMEM((16,), jnp.int32),
            pltpu.VMEM((16,), jnp.int32),
            pltpu.VMEM((256,), jnp.int32),
            pltpu.VMEM((256,), jnp.int32),
            pltpu.VMEM((256,), jnp.int32),
            pltpu.VMEM((256,), jnp.int32),
            pltpu.VMEM((128,), jnp.int32),
            pltpu.VMEM((128,), jnp.int32),
            pltpu.VMEM((_ECHUNK, D), jnp.float32),
            pltpu.VMEM((_ECHUNK, D), jnp.float32),
            pltpu.VMEM((_ECHUNK, D), jnp.float32),
            pltpu.VMEM_SHARED((slab + _ECHUNK, D), jnp.float32),
            pltpu.SemaphoreType.DMA,
            pltpu.SemaphoreType.DMA,
            pltpu.SemaphoreType.DMA,
            pltpu.SemaphoreType.DMA,
            pltpu.SemaphoreType.DMA,
        ],
    )
    zeros128 = jnp.zeros((_ECHUNK, D), jnp.float32)
    return f(node_pad, edge_pad, lsrc, perm, lldst, obs, obm, zeros128)


def _graph_lists(src, dst, n_nodes, nb, shift):
    slab = 1 << shift
    npad = _EDGE_PAD - src.shape[0]
    ar = jnp.arange(npad, dtype=jnp.int32)
    src_f = jnp.concatenate([src.astype(jnp.int32), ar % n_nodes])
    dst_f = jnp.concatenate([dst.astype(jnp.int32),
                             n_nodes + ar % (nb * slab - n_nodes)])
    counts = _count_call(dst_f.reshape(-1, 1), shift)[0].astype(jnp.int32)
    off = jnp.concatenate([jnp.zeros((1,), jnp.int32), jnp.cumsum(counts)])
    bases = jnp.zeros((8, 128), jnp.int32).at[0, :65].set(off[:65])
    pos = _pos_call(dst_f.reshape(-1, 1), bases, shift)
    padbase = off[nb]
    padv = jnp.broadcast_to(padbase.reshape(1), (16,)).astype(jnp.int32)
    perm, lsrc, lldst = _sc_permute(pos.reshape(-1), src_f, dst_f, padv, slab)
    npasses = (nb + _NC - 1) // _NC
    nch = (off[1:] - off[:-1] + 255) // 256
    starts = off[:nb]
    nchb = nch[:nb]
    segl = (off[1:] - off[:-1])[:nb]
    validb = jnp.ones((nb,), jnp.int32)
    if nb % _NC:
        starts = jnp.concatenate([starts, jnp.zeros((1,), jnp.int32)])
        nchb = jnp.concatenate([nchb, jnp.zeros((1,), jnp.int32)])
        segl = jnp.concatenate([segl, jnp.zeros((1,), jnp.int32)])
        validb = jnp.concatenate([validb, jnp.zeros((1,), jnp.int32)])
    obs = jnp.zeros((32, _NC, 16), jnp.int32)
    obs = obs.at[:npasses].set(
        jnp.broadcast_to(starts.reshape(npasses, _NC, 1), (npasses, _NC, 16)))
    obm = jnp.zeros((32, _NC, 16), jnp.int32)
    obm = obm.at[:npasses, :, 0].set(nchb.reshape(npasses, _NC))
    obm = obm.at[:npasses, :, 1].set(validb.reshape(npasses, _NC))
    obm = obm.at[:npasses, :, 2].set(segl.reshape(npasses, _NC))
    obm = obm.at[:npasses, :, 3].set(jnp.broadcast_to(
        padbase.reshape(1, 1), (npasses, _NC)).astype(jnp.int32))
    return lsrc, perm, lldst, obs, obm


def kernel(AtomBondGraph_edges, BondAngleGraph_edges, AngleDihedralGraph_edges,
           x, bond_attr, bond_lengths, bond_angles, dihedral_angles,
           atom_batch, num_bonds, num_angles, num_graphs,
           masked_atom_indices, masked_bond_indices, masked_angle_indices,
           masked_dihedral_indices, params):
    _x = x.at[masked_atom_indices].set(15)
    _battr = bond_attr.at[masked_bond_indices].set(7)
    _bl = bond_lengths.at[masked_bond_indices].set(0.0)
    _ang = bond_angles.at[masked_angle_indices].set(0.0)
    _dih = dihedral_angles.at[masked_dihedral_indices].set(0.0)

    sb = jnp.searchsorted(atom_batch, jnp.arange(NUM_GRAPHS + 1, dtype=atom_batch.dtype))
    atom_counts = (sb[1:] - sb[:-1]).astype(jnp.float32)
    inv_atoms = jax.lax.rsqrt(jnp.maximum(atom_counts, 1.0)).reshape(NUM_GRAPHS, 1)
    bond_counts = num_bonds.astype(jnp.float32)
    inv_bonds = jax.lax.rsqrt(jnp.maximum(bond_counts, 1.0)).reshape(NUM_GRAPHS, 1)
    angle_counts = num_angles.astype(jnp.float32)
    inv_angles = jax.lax.rsqrt(jnp.maximum(angle_counts, 1.0)).reshape(NUM_GRAPHS, 1)

    gid = jnp.arange(NUM_GRAPHS, dtype=jnp.int32)
    bond_batch = jnp.repeat(gid, num_bonds, total_repeat_length=N_BONDS)
    angle_batch = jnp.repeat(gid, num_angles, total_repeat_length=N_ANGLES)
    ab_pad = _pad_rows(atom_batch.astype(jnp.int32).reshape(-1, 1), _ATOM_PAD, -1)
    bb_pad = _pad_rows(bond_batch.reshape(-1, 1), _EDGE_PAD, -1)
    anb_pad = _pad_rows(angle_batch.reshape(-1, 1), _EDGE_PAD, -1)

    atom_tab = jnp.concatenate(params["atom_emb"], axis=0)
    x_pad = _pad_rows(_x.astype(jnp.int32), _ATOM_PAD)
    node_hidden = _embed_call(x_pad, atom_tab, 16, 7, None, _ATOM_PAD)[:N_ATOMS]

    def bond_feat_input():
        battr_pad = _pad_rows(_battr.astype(jnp.int32), _EDGE_PAD)
        bl_bits = jax.lax.bitcast_convert_type(
            _bl.astype(jnp.float32), jnp.int32).reshape(-1, 1)
        return jnp.concatenate([battr_pad, _pad_rows(bl_bits, _EDGE_PAD)], axis=1)

    bond_feats = bond_feat_input()

    def bond_embed(tables, rbf_p):
        tab = jnp.concatenate(list(tables) + [rbf_p["W"]], axis=0)
        out = _embed_call(bond_feats, tab, 8, 3, _BL_CENTERS, _EDGE_PAD)
        return out + rbf_p["b"][None, :]

    bond_hidden = bond_embed(params["init_bond_emb"], params["init_bond_rbf"])[:N_BONDS]

    ang_pad = _pad_rows(_ang.astype(jnp.float32), _EDGE_PAD)
    dih_pad = _pad_rows(_dih.astype(jnp.float32), _EDGE_PAD)
    angle_hidden = _rbf_call(ang_pad, params["init_angle_rbf"]["W"],
                             params["init_angle_rbf"]["b"], _BA_CENTERS,
                             _EDGE_PAD)[:N_ANGLES]

    npad_ab = _EDGE_PAD - N_BONDS
    ar_ab = jnp.arange(npad_ab, dtype=jnp.int32)
    ab_src_f = jnp.concatenate(
        [AtomBondGraph_edges[0].astype(jnp.int32), ar_ab % N_ATOMS])
    ab_dst_f = jnp.concatenate(
        [AtomBondGraph_edges[1].astype(jnp.int32),
         N_ATOMS + ar_ab % (_ATOM_PAD - N_ATOMS)])
    ba_g = _graph_lists(BondAngleGraph_edges[0], BondAngleGraph_edges[1],
                        N_BONDS, 40, 13)
    ad_g = _graph_lists(AngleDihedralGraph_edges[0], AngleDihedralGraph_edges[1],
                        N_ANGLES, 40, 13)

    nh_pad = _pad_rows(node_hidden, _ATOM_PAD)
    eh_pad = _pad_rows(bond_hidden, _EDGE_PAD)
    ah_pad = _pad_rows(angle_hidden, _EDGE_PAD)
    dih_hidden = None
    for lid in range(N_LAYERS):
        lp = params["layers"][lid]
        act = lid != N_LAYERS - 1

        agg_a = _sc_atoms(nh_pad, eh_pad, ab_src_f, ab_dst_f)
        nh_pad = _block_dense_call(
            agg_a, nh_pad, ab_pad, inv_atoms, lp["ab_block"], act, _ATOM_PAD)

        cur_edge_pad = bond_embed(lp["bond_emb"], lp["bond_rbf"])
        agg_b = _sc_consume(cur_edge_pad, ah_pad, ba_g[0], ba_g[1], ba_g[2],
                            ba_g[3], ba_g[4], 40, 8192)
        eh_pad = _block_dense_call(
            agg_b, cur_edge_pad, bb_pad,
            inv_bonds, lp["ba_block"], act, _EDGE_PAD)

        cur_angle_pad = _rbf_call(ang_pad, lp["angle_rbf"]["W"],
                                  lp["angle_rbf"]["b"], _BA_CENTERS, _EDGE_PAD)
        dih_hidden = _rbf_call(dih_pad, lp["dihedral_rbf"]["W"],
                               lp["dihedral_rbf"]["b"], _DA_CENTERS, _EDGE_PAD)
        agg_an = _sc_consume(cur_angle_pad, dih_hidden, ad_g[0], ad_g[1],
                             ad_g[2], ad_g[3], ad_g[4], 40, 8192)
        ah_pad = _block_dense_call(
            agg_an, cur_angle_pad,
            anb_pad, inv_angles, lp["ad_block"], act, _EDGE_PAD)

    graph_repr = _graph_repr_call(nh_pad, ab_pad,
                                  atom_counts.reshape(NUM_GRAPHS, 1))
    return (nh_pad[:N_ATOMS], eh_pad[:N_BONDS], ah_pad[:N_ANGLES],
            dih_hidden[:N_DIHEDRALS], graph_repr)
